# baseline (speedup 1.0000x reference)
import functools

import jax
import jax.numpy as jnp
from jax.experimental import pallas as pl
from jax.experimental.pallas import tpu as pltpu
from jax.experimental.pallas import tpu_sc as plsc

_NUM_CLASSES = 8
_CONF = 0.05
_IOU_T = 0.5
_MAX_DET = 100
_MAX_DET_PER_CLASS = 100

_N = 20000
_NPAD = 20480
_LANES = 128

_SLOT_ROWS = 168
_SLOT = _SLOT_ROWS * _LANES
_FAST_ROWS = 32
_CHUNK = 1280
_NCHUNKS = _NPAD // _CHUNK


def _compact_kernel(cls_hbm, x1_hbm, y1_hbm, x2_hbm, y2_hbm, sc_hbm,
                    ox1, oy1, ox2, oy2, osc, ocnt,
                    cls_v, px1, py1, px2, py2, psc,
                    st_x1, st_y1, st_x2, st_y2, st_sc, cnt_v):
    b = jax.lax.axis_index("c")
    k = jax.lax.axis_index("s")

    @pl.when(k < _NUM_CLASSES)
    def _():
        myc = k

        def zbody(j, carry):
            st_sc[pl.ds(j * 16, 16)] = jnp.zeros((16,), jnp.float32)
            return carry

        jax.lax.fori_loop(0, (_SLOT + 16) // 16, zbody, jnp.int32(0))

        def chunk_body(t, cursor):
            off = pl.multiple_of(b * _NPAD + t * _CHUNK, 8)
            pltpu.sync_copy(cls_hbm.at[pl.ds(off, _CHUNK)], cls_v)
            pltpu.sync_copy(x1_hbm.at[pl.ds(off, _CHUNK)], px1)
            pltpu.sync_copy(y1_hbm.at[pl.ds(off, _CHUNK)], py1)
            pltpu.sync_copy(x2_hbm.at[pl.ds(off, _CHUNK)], px2)
            pltpu.sync_copy(y2_hbm.at[pl.ds(off, _CHUNK)], py2)
            pltpu.sync_copy(sc_hbm.at[pl.ds(off, _CHUNK)], psc)

            ones16 = jnp.full((16,), 1, jnp.int32)
            zeros16 = jnp.full((16,), 0, jnp.int32)
            lane16 = jnp.arange(16, dtype=jnp.int32)
            trash16 = lane16 + jnp.full((16,), _SLOT, jnp.int32)
            mycv = jnp.full((16,), myc, jnp.int32)
            sixteen16 = jnp.full((16,), 16, jnp.int32)

            def vec_body(i, curv):
                v = cls_v[pl.ds(i * 16, 16)]
                mask = v == mycv
                pop = plsc.all_reduce_population_count(mask)
                keys = jnp.where(mask, lane16, lane16 + sixteen16)
                _, sval = plsc.sort_key_val(keys, lane16)
                srcdst = jnp.where(lane16 < pop, curv + lane16, trash16)
                plsc.store_scatter(cnt_v, [sval], srcdst)
                dst = cnt_v[...]
                plsc.store_scatter(st_x1, [dst], px1[pl.ds(i * 16, 16)])
                plsc.store_scatter(st_y1, [dst], py1[pl.ds(i * 16, 16)])
                plsc.store_scatter(st_x2, [dst], px2[pl.ds(i * 16, 16)])
                plsc.store_scatter(st_y2, [dst], py2[pl.ds(i * 16, 16)])
                plsc.store_scatter(st_sc, [dst], psc[pl.ds(i * 16, 16)])
                return curv + pop

            return jax.lax.fori_loop(0, _CHUNK // 16, vec_body, cursor)

        n_v = jax.lax.fori_loop(0, _NCHUNKS, chunk_body,
                                jnp.full((16,), 0, jnp.int32))
        npad_v = ((n_v + jnp.full((16,), 7, jnp.int32))
                  & jnp.full((16,), -8, jnp.int32))

        slot = pl.multiple_of((b * _NUM_CLASSES + myc) * _SLOT, 8)
        pltpu.sync_copy(st_x1.at[pl.ds(0, _SLOT)], ox1.at[pl.ds(slot, _SLOT)])
        pltpu.sync_copy(st_y1.at[pl.ds(0, _SLOT)], oy1.at[pl.ds(slot, _SLOT)])
        pltpu.sync_copy(st_x2.at[pl.ds(0, _SLOT)], ox2.at[pl.ds(slot, _SLOT)])
        pltpu.sync_copy(st_y2.at[pl.ds(0, _SLOT)], oy2.at[pl.ds(slot, _SLOT)])
        pltpu.sync_copy(st_sc.at[pl.ds(0, _SLOT)], osc.at[pl.ds(slot, _SLOT)])

        iota16 = jnp.arange(16, dtype=jnp.int32)
        cnt_v[...] = jnp.where(iota16 == 0, npad_v,
                               jnp.full((16,), 0, jnp.int32))
        coff = pl.multiple_of((b * _NUM_CLASSES + myc) * 16, 8)
        pltpu.sync_copy(cnt_v, ocnt.at[pl.ds(coff, 16)])


def _sc_compact(cls_f, x1f, y1f, x2f, y2f, scf):
    pay = jax.ShapeDtypeStruct((2 * _NUM_CLASSES * _SLOT,), jnp.float32)
    kfn = pl.kernel(
        _compact_kernel,
        mesh=plsc.VectorSubcoreMesh(core_axis_name="c", subcore_axis_name="s"),
        compiler_params=pltpu.CompilerParams(needs_layout_passes=False),
        out_type=[pay] * 5
        + [jax.ShapeDtypeStruct((2 * _NUM_CLASSES * 16,), jnp.int32)],
        scratch_types=[pltpu.VMEM((_CHUNK,), jnp.int32)]
        + [pltpu.VMEM((_CHUNK,), jnp.float32)] * 5
        + [pltpu.VMEM((_SLOT + 16,), jnp.float32)] * 5
        + [pltpu.VMEM((16,), jnp.int32)],
    )
    return kfn(cls_f, x1f, y1f, x2f, y2f, scf)


def _nms_kernel(x1, y1, x2, y2, sc, cf,
                sel_s, sel_y1, sel_x1, sel_y2, sel_x2,
                ny1, nx1, ny2, nx2, a2s, ss):
    C = _NUM_CLASSES
    lane = jax.lax.broadcasted_iota(jnp.int32, (1, _LANES), 1)
    cfv = cf[0]
    tot = [jnp.sum(jnp.where(lane == c, cfv, 0.0)).astype(jnp.int32)
           for c in range(C)]
    maxt = tot[0]
    for c in range(1, C):
        maxt = jnp.maximum(maxt, tot[c])

    sel_s[0] = jnp.full((C, _LANES), -jnp.inf, jnp.float32)
    zeros = jnp.zeros((C, _LANES), jnp.float32)
    sel_y1[0] = zeros
    sel_x1[0] = zeros
    sel_y2[0] = zeros
    sel_x2[0] = zeros

    def run(R):
        flat = (jax.lax.broadcasted_iota(jnp.int32, (R, _LANES), 0) * _LANES
                + jax.lax.broadcasted_iota(jnp.int32, (R, _LANES), 1))
        for c in range(C):
            base = c * _SLOT_ROWS
            ny1v = y1[0, pl.ds(base, R), :] / 512.0
            nx1v = x1[0, pl.ds(base, R), :] / 512.0
            ny2v = y2[0, pl.ds(base, R), :] / 512.0
            nx2v = x2[0, pl.ds(base, R), :] / 512.0
            ny1[pl.ds(base, R), :] = ny1v
            nx1[pl.ds(base, R), :] = nx1v
            ny2[pl.ds(base, R), :] = ny2v
            nx2[pl.ds(base, R), :] = nx2v
            a2s[pl.ds(base, R), :] = (ny2v - ny1v) * (nx2v - nx1v)
            scv = sc[0, pl.ds(base, R), :]
            ss[pl.ds(base, R), :] = jnp.where(
                (flat < tot[c]) & (scv > _CONF), scv, -jnp.inf)

        m_init = tuple(jnp.max(ss[pl.ds(c * _SLOT_ROWS, R), :])
                       for c in range(C))

        def cond(carry):
            step = carry[0]
            ms = carry[1:]
            any_m = ms[0]
            for c in range(1, C):
                any_m = jnp.maximum(any_m, ms[c])
            return (step < _MAX_DET_PER_CLASS) & (any_m > _CONF)

        def body(carry):
            step = carry[0]
            ms = carry[1:]
            act = [ms[c] > _CONF for c in range(C)]
            sv = [ss[pl.ds(c * _SLOT_ROWS, R), :] for c in range(C)]
            idx = [None] * C
            for c in range(C):
                eq = sv[c] == ms[c]
                idx[c] = jnp.min(jnp.where(eq, flat, jnp.int32(2 ** 30)))
            row, colmask = [None] * C, [None] * C
            for c in range(C):
                i = jnp.where(act[c], idx[c], 0)
                idx[c] = i
                row[c] = c * _SLOT_ROWS + i // _LANES
                colmask[c] = lane == (i % _LANES)
            by1, bx1, by2, bx2 = [None] * C, [None] * C, [None] * C, [None] * C
            for c in range(C):
                by1[c] = jnp.sum(jnp.where(colmask[c], ny1[pl.ds(row[c], 1), :], 0.0))
                bx1[c] = jnp.sum(jnp.where(colmask[c], nx1[pl.ds(row[c], 1), :], 0.0))
                by2[c] = jnp.sum(jnp.where(colmask[c], ny2[pl.ds(row[c], 1), :], 0.0))
                bx2[c] = jnp.sum(jnp.where(colmask[c], nx2[pl.ds(row[c], 1), :], 0.0))
            for c in range(C):
                lm = (lane == step) & act[c]
                sel_s[0, pl.ds(c, 1), :] = jnp.where(lm, ms[c], sel_s[0, pl.ds(c, 1), :])
                sel_y1[0, pl.ds(c, 1), :] = jnp.where(lm, by1[c], sel_y1[0, pl.ds(c, 1), :])
                sel_x1[0, pl.ds(c, 1), :] = jnp.where(lm, bx1[c], sel_x1[0, pl.ds(c, 1), :])
                sel_y2[0, pl.ds(c, 1), :] = jnp.where(lm, by2[c], sel_y2[0, pl.ds(c, 1), :])
                sel_x2[0, pl.ds(c, 1), :] = jnp.where(lm, bx2[c], sel_x2[0, pl.ds(c, 1), :])
            new_ms = []
            for c in range(C):
                base = c * _SLOT_ROWS
                yy1 = jnp.maximum(by1[c], ny1[pl.ds(base, R), :])
                xx1 = jnp.maximum(bx1[c], nx1[pl.ds(base, R), :])
                yy2 = jnp.minimum(by2[c], ny2[pl.ds(base, R), :])
                xx2 = jnp.minimum(bx2[c], nx2[pl.ds(base, R), :])
                inter = jnp.maximum(yy2 - yy1, 0.0) * jnp.maximum(xx2 - xx1, 0.0)
                a1 = (by2[c] - by1[c]) * (bx2[c] - bx1[c])
                iou = inter / (a1 + a2s[pl.ds(base, R), :] - inter + 1e-8)
                snew = jnp.where(iou > _IOU_T, -jnp.inf, sv[c])
                ss[pl.ds(base, R), :] = snew
                new_ms.append(jnp.max(snew))
            return (step + 1,) + tuple(new_ms)

        jax.lax.while_loop(cond, body, (jnp.int32(0),) + m_init)

    @pl.when(maxt <= _FAST_ROWS * _LANES)
    def _():
        run(_FAST_ROWS)

    @pl.when(maxt > _FAST_ROWS * _LANES)
    def _():
        run(_SLOT_ROWS)


def _merge_kernel(ms, my1, mx1, my2, mx2, res, scr):
    crow = jax.lax.broadcasted_iota(jnp.int32, (_NUM_CLASSES, _LANES), 0)
    lane = jax.lax.broadcasted_iota(jnp.int32, (_NUM_CLASSES, _LANES), 1)
    lane1 = jax.lax.broadcasted_iota(jnp.int32, (1, _LANES), 1)
    validlane = lane < _MAX_DET_PER_CLASS
    flat = jnp.where(validlane, crow * _MAX_DET_PER_CLASS + lane,
                     jnp.int32(2 ** 30))

    scr[...] = jnp.where(validlane, ms[0], -jnp.inf)
    res[0] = jnp.zeros((_NUM_CLASSES, _LANES), jnp.float32)

    m0 = jnp.max(scr[...])

    def cond(carry):
        step, m = carry
        return (step < _MAX_DET) & (m > _CONF)

    def body(carry):
        step, m = carry
        sv = scr[...]
        eq = sv == m
        fidx = jnp.min(jnp.where(eq, flat, jnp.int32(2 ** 30)))
        c = fidx // _MAX_DET_PER_CLASS
        j = fidx % _MAX_DET_PER_CLASS
        mask = (crow == c) & (lane == j)
        by1 = jnp.sum(jnp.where(mask, my1[0], 0.0))
        bx1 = jnp.sum(jnp.where(mask, mx1[0], 0.0))
        by2 = jnp.sum(jnp.where(mask, my2[0], 0.0))
        bx2 = jnp.sum(jnp.where(mask, mx2[0], 0.0))

        lm = lane1 == step
        res[0, pl.ds(0, 1), :] = jnp.where(lm, bx1 * 512.0, res[0, pl.ds(0, 1), :])
        res[0, pl.ds(1, 1), :] = jnp.where(lm, by1 * 512.0, res[0, pl.ds(1, 1), :])
        res[0, pl.ds(2, 1), :] = jnp.where(lm, bx2 * 512.0, res[0, pl.ds(2, 1), :])
        res[0, pl.ds(3, 1), :] = jnp.where(lm, by2 * 512.0, res[0, pl.ds(3, 1), :])
        res[0, pl.ds(4, 1), :] = jnp.where(lm, c.astype(jnp.float32), res[0, pl.ds(4, 1), :])
        res[0, pl.ds(5, 1), :] = jnp.where(lm, m, res[0, pl.ds(5, 1), :])

        snew = jnp.where(mask, -jnp.inf, sv)
        scr[...] = snew
        return step + 1, jnp.max(snew)

    nstep, _ = jax.lax.while_loop(cond, body, (jnp.int32(0), m0))
    res[0, pl.ds(6, 1), :] = jnp.where(lane1 == 0, nstep.astype(jnp.float32),
                                       res[0, pl.ds(6, 1), :])


def _nms_from_compact(X1, Y1, X2, Y2, SC, cf):
    B = X1.shape[0]
    pay_spec = pl.BlockSpec((1, _NUM_CLASSES * _SLOT_ROWS, _LANES),
                            lambda b: (b, 0, 0))
    cf_spec = pl.BlockSpec((1, 1, _LANES), lambda b: (b, 0, 0))
    out_spec = pl.BlockSpec((1, _NUM_CLASSES, _LANES), lambda b: (b, 0, 0))
    out_shape = jax.ShapeDtypeStruct((B, _NUM_CLASSES, _LANES), jnp.float32)
    big = (_NUM_CLASSES * _SLOT_ROWS, _LANES)

    sel_s, sel_y1, sel_x1, sel_y2, sel_x2 = pl.pallas_call(
        _nms_kernel,
        grid=(B,),
        in_specs=[pay_spec] * 5 + [cf_spec],
        out_specs=[out_spec] * 5,
        out_shape=[out_shape] * 5,
        scratch_shapes=[pltpu.VMEM(big, jnp.float32)] * 6,
        compiler_params=pltpu.CompilerParams(
            dimension_semantics=("parallel",)),
    )(X1, Y1, X2, Y2, SC, cf)

    mspec = pl.BlockSpec((1, _NUM_CLASSES, _LANES), lambda b: (b, 0, 0))
    res = pl.pallas_call(
        _merge_kernel,
        grid=(B,),
        in_specs=[mspec] * 5,
        out_specs=mspec,
        out_shape=jax.ShapeDtypeStruct((B, _NUM_CLASSES, _LANES), jnp.float32),
        scratch_shapes=[pltpu.VMEM((_NUM_CLASSES, _LANES), jnp.float32)],
    )(sel_s, sel_y1, sel_x1, sel_y2, sel_x2)

    out6 = jnp.transpose(res[:, 0:6, 0:_MAX_DET], (0, 2, 1))
    valid_det = res[:, 6, 0].astype(jnp.int32)
    return out6, valid_det


@jax.jit
def kernel(images, predictions):
    B = predictions.shape[0]

    def _flat(a, pad_value):
        a = jnp.pad(a, ((0, 0), (0, _NPAD - _N)), constant_values=pad_value)
        return a.reshape(B * _NPAD)

    x1f = _flat(predictions[..., 0], 0.0)
    y1f = _flat(predictions[..., 1], 0.0)
    x2f = _flat(predictions[..., 2], 0.0)
    y2f = _flat(predictions[..., 3], 0.0)
    clsf = _flat(predictions[..., 4].astype(jnp.int32), _NUM_CLASSES)
    scf = _flat(predictions[..., 5], 0.0)

    ox1, oy1, ox2, oy2, osc, ocnt = _sc_compact(clsf, x1f, y1f, x2f, y2f, scf)

    shp = (B, _NUM_CLASSES * _SLOT_ROWS, _LANES)
    X1 = ox1.reshape(shp)
    Y1 = oy1.reshape(shp)
    X2 = ox2.reshape(shp)
    Y2 = oy2.reshape(shp)
    SCp = osc.reshape(shp)
    cnts = ocnt.reshape(B, _NUM_CLASSES, 16)[:, :, 0].astype(jnp.float32)
    cf = jnp.zeros((B, _LANES), jnp.float32).at[:, :_NUM_CLASSES].set(cnts)
    cf = cf.reshape(B, 1, _LANES)

    return _nms_from_compact(X1, Y1, X2, Y2, SCp, cf)


# speedup vs baseline: 1.2102x; 1.2102x over previous
import functools

import jax
import jax.numpy as jnp
from jax.experimental import pallas as pl
from jax.experimental.pallas import tpu as pltpu
from jax.experimental.pallas import tpu_sc as plsc

_NUM_CLASSES = 8
_CONF = 0.05
_IOU_T = 0.5
_MAX_DET = 100
_MAX_DET_PER_CLASS = 100

_N = 20000
_NPAD = 20480
_LANES = 128

_SLOT_ROWS = 168
_SLOT = _SLOT_ROWS * _LANES
_FAST_ROWS = 32
_CHUNK = 1280
_NCHUNKS = _NPAD // _CHUNK


def _compact_kernel(cls_hbm, x1_hbm, y1_hbm, x2_hbm, y2_hbm, sc_hbm,
                    ox1, oy1, ox2, oy2, osc, ocnt,
                    cls_v, px1, py1, px2, py2, psc,
                    cls_w, qx1, qy1, qx2, qy2, qsc,
                    st_x1, st_y1, st_x2, st_y2, st_sc, cnt_v,
                    sem_a, sem_b):
    b = jax.lax.axis_index("c")
    k = jax.lax.axis_index("s")

    @pl.when(k < _NUM_CLASSES)
    def _():
        myc = k

        def zbody(j, carry):
            st_sc[pl.ds(j * 16, 16)] = jnp.zeros((16,), jnp.float32)
            return carry

        jax.lax.fori_loop(0, (_SLOT + 16) // 16, zbody, jnp.int32(0))

        ones16 = jnp.full((16,), 1, jnp.int32)
        zeros16 = jnp.full((16,), 0, jnp.int32)
        lane16 = jnp.arange(16, dtype=jnp.int32)
        trash16 = lane16 + jnp.full((16,), _SLOT, jnp.int32)
        mycv = jnp.full((16,), myc, jnp.int32)
        sixteen16 = jnp.full((16,), 16, jnp.int32)

        bufs_a = (cls_v, px1, py1, px2, py2, psc)
        bufs_b = (cls_w, qx1, qy1, qx2, qy2, qsc)
        srcs = (cls_hbm, x1_hbm, y1_hbm, x2_hbm, y2_hbm, sc_hbm)

        def copies(t, bufs, sem):
            off = pl.multiple_of(b * _NPAD + t * _CHUNK, 8)
            return [pltpu.make_async_copy(s.at[pl.ds(off, _CHUNK)], d, sem)
                    for s, d in zip(srcs, bufs)]

        def issue(t, bufs, sem):
            for c in copies(t, bufs, sem):
                c.start()

        def wait(t, bufs, sem):
            for c in copies(t, bufs, sem):
                c.wait()

        def process(bufs, curv):
            cv, bx1, by1, bx2, by2, bsc = bufs

            def vec4(j, cur):
                for u in range(4):
                    i = j * 4 + u
                    v = cv[pl.ds(i * 16, 16)]
                    mask = v == mycv
                    pop = plsc.all_reduce_population_count(mask)
                    keys = jnp.where(mask, lane16, lane16 + sixteen16)
                    _, sval = plsc.sort_key_val(keys, lane16)
                    srcdst = jnp.where(lane16 < pop, cur + lane16, trash16)
                    plsc.store_scatter(cnt_v, [sval], srcdst)
                    dst = cnt_v[...]
                    plsc.store_scatter(st_x1, [dst], bx1[pl.ds(i * 16, 16)])
                    plsc.store_scatter(st_y1, [dst], by1[pl.ds(i * 16, 16)])
                    plsc.store_scatter(st_x2, [dst], bx2[pl.ds(i * 16, 16)])
                    plsc.store_scatter(st_y2, [dst], by2[pl.ds(i * 16, 16)])
                    plsc.store_scatter(st_sc, [dst], bsc[pl.ds(i * 16, 16)])
                    cur = cur + pop
                return cur

            return jax.lax.fori_loop(0, _CHUNK // 64, vec4, curv)

        issue(0, bufs_a, sem_a)

        def outer(t, curv):
            t2 = t * 2
            issue(t2 + 1, bufs_b, sem_b)
            wait(t2, bufs_a, sem_a)
            curv = process(bufs_a, curv)

            @pl.when(t < _NCHUNKS // 2 - 1)
            def _():
                issue(t2 + 2, bufs_a, sem_a)

            wait(t2 + 1, bufs_b, sem_b)
            curv = process(bufs_b, curv)
            return curv

        n_v = jax.lax.fori_loop(0, _NCHUNKS // 2, outer,
                                jnp.full((16,), 0, jnp.int32))
        npad_v = ((n_v + jnp.full((16,), 7, jnp.int32))
                  & jnp.full((16,), -8, jnp.int32))

        slot = pl.multiple_of((b * _NUM_CLASSES + myc) * _SLOT, 8)
        pltpu.sync_copy(st_x1.at[pl.ds(0, _SLOT)], ox1.at[pl.ds(slot, _SLOT)])
        pltpu.sync_copy(st_y1.at[pl.ds(0, _SLOT)], oy1.at[pl.ds(slot, _SLOT)])
        pltpu.sync_copy(st_x2.at[pl.ds(0, _SLOT)], ox2.at[pl.ds(slot, _SLOT)])
        pltpu.sync_copy(st_y2.at[pl.ds(0, _SLOT)], oy2.at[pl.ds(slot, _SLOT)])
        pltpu.sync_copy(st_sc.at[pl.ds(0, _SLOT)], osc.at[pl.ds(slot, _SLOT)])

        iota16 = jnp.arange(16, dtype=jnp.int32)
        cnt_v[...] = jnp.where(iota16 == 0, npad_v,
                               jnp.full((16,), 0, jnp.int32))
        coff = pl.multiple_of((b * _NUM_CLASSES + myc) * 16, 8)
        pltpu.sync_copy(cnt_v, ocnt.at[pl.ds(coff, 16)])


def _sc_compact(cls_f, x1f, y1f, x2f, y2f, scf):
    pay = jax.ShapeDtypeStruct((2 * _NUM_CLASSES * _SLOT,), jnp.float32)
    kfn = pl.kernel(
        _compact_kernel,
        mesh=plsc.VectorSubcoreMesh(core_axis_name="c", subcore_axis_name="s"),
        compiler_params=pltpu.CompilerParams(needs_layout_passes=False),
        out_type=[pay] * 5
        + [jax.ShapeDtypeStruct((2 * _NUM_CLASSES * 16,), jnp.int32)],
        scratch_types=([pltpu.VMEM((_CHUNK,), jnp.int32)]
                       + [pltpu.VMEM((_CHUNK,), jnp.float32)] * 5) * 2
        + [pltpu.VMEM((_SLOT + 16,), jnp.float32)] * 5
        + [pltpu.VMEM((16,), jnp.int32)]
        + [pltpu.SemaphoreType.DMA] * 2,
    )
    return kfn(cls_f, x1f, y1f, x2f, y2f, scf)


def _nms_kernel(x1, y1, x2, y2, sc, cf,
                sel_s, sel_y1, sel_x1, sel_y2, sel_x2,
                ny1, nx1, ny2, nx2, a2s, ss):
    C = _NUM_CLASSES
    lane = jax.lax.broadcasted_iota(jnp.int32, (1, _LANES), 1)
    cfv = cf[0]
    tot = [jnp.sum(jnp.where(lane == c, cfv, 0.0)).astype(jnp.int32)
           for c in range(C)]
    maxt = tot[0]
    for c in range(1, C):
        maxt = jnp.maximum(maxt, tot[c])

    sel_s[0] = jnp.full((C, _LANES), -jnp.inf, jnp.float32)
    zeros = jnp.zeros((C, _LANES), jnp.float32)
    sel_y1[0] = zeros
    sel_x1[0] = zeros
    sel_y2[0] = zeros
    sel_x2[0] = zeros

    def run(R):
        flat = (jax.lax.broadcasted_iota(jnp.int32, (R, _LANES), 0) * _LANES
                + jax.lax.broadcasted_iota(jnp.int32, (R, _LANES), 1))
        for c in range(C):
            base = c * _SLOT_ROWS
            ny1v = y1[0, pl.ds(base, R), :] / 512.0
            nx1v = x1[0, pl.ds(base, R), :] / 512.0
            ny2v = y2[0, pl.ds(base, R), :] / 512.0
            nx2v = x2[0, pl.ds(base, R), :] / 512.0
            ny1[pl.ds(base, R), :] = ny1v
            nx1[pl.ds(base, R), :] = nx1v
            ny2[pl.ds(base, R), :] = ny2v
            nx2[pl.ds(base, R), :] = nx2v
            a2s[pl.ds(base, R), :] = (ny2v - ny1v) * (nx2v - nx1v)
            scv = sc[0, pl.ds(base, R), :]
            ss[pl.ds(base, R), :] = jnp.where(
                (flat < tot[c]) & (scv > _CONF), scv, -jnp.inf)

        m_init = tuple(jnp.max(ss[pl.ds(c * _SLOT_ROWS, R), :])
                       for c in range(C))

        def cond(carry):
            step = carry[0]
            ms = carry[1:]
            any_m = ms[0]
            for c in range(1, C):
                any_m = jnp.maximum(any_m, ms[c])
            return (step < _MAX_DET_PER_CLASS) & (any_m > _CONF)

        def body(carry):
            step = carry[0]
            ms = carry[1:]
            act = [ms[c] > _CONF for c in range(C)]
            sv = [ss[pl.ds(c * _SLOT_ROWS, R), :] for c in range(C)]
            idx = [None] * C
            for c in range(C):
                eq = sv[c] == ms[c]
                idx[c] = jnp.min(jnp.where(eq, flat, jnp.int32(2 ** 30)))
            row, colmask = [None] * C, [None] * C
            for c in range(C):
                i = jnp.where(act[c], idx[c], 0)
                idx[c] = i
                row[c] = c * _SLOT_ROWS + i // _LANES
                colmask[c] = lane == (i % _LANES)
            by1, bx1, by2, bx2 = [None] * C, [None] * C, [None] * C, [None] * C
            for c in range(C):
                by1[c] = jnp.sum(jnp.where(colmask[c], ny1[pl.ds(row[c], 1), :], 0.0))
                bx1[c] = jnp.sum(jnp.where(colmask[c], nx1[pl.ds(row[c], 1), :], 0.0))
                by2[c] = jnp.sum(jnp.where(colmask[c], ny2[pl.ds(row[c], 1), :], 0.0))
                bx2[c] = jnp.sum(jnp.where(colmask[c], nx2[pl.ds(row[c], 1), :], 0.0))
            for c in range(C):
                lm = (lane == step) & act[c]
                sel_s[0, pl.ds(c, 1), :] = jnp.where(lm, ms[c], sel_s[0, pl.ds(c, 1), :])
                sel_y1[0, pl.ds(c, 1), :] = jnp.where(lm, by1[c], sel_y1[0, pl.ds(c, 1), :])
                sel_x1[0, pl.ds(c, 1), :] = jnp.where(lm, bx1[c], sel_x1[0, pl.ds(c, 1), :])
                sel_y2[0, pl.ds(c, 1), :] = jnp.where(lm, by2[c], sel_y2[0, pl.ds(c, 1), :])
                sel_x2[0, pl.ds(c, 1), :] = jnp.where(lm, bx2[c], sel_x2[0, pl.ds(c, 1), :])
            new_ms = []
            for c in range(C):
                base = c * _SLOT_ROWS
                yy1 = jnp.maximum(by1[c], ny1[pl.ds(base, R), :])
                xx1 = jnp.maximum(bx1[c], nx1[pl.ds(base, R), :])
                yy2 = jnp.minimum(by2[c], ny2[pl.ds(base, R), :])
                xx2 = jnp.minimum(bx2[c], nx2[pl.ds(base, R), :])
                inter = jnp.maximum(yy2 - yy1, 0.0) * jnp.maximum(xx2 - xx1, 0.0)
                a1 = (by2[c] - by1[c]) * (bx2[c] - bx1[c])
                iou = inter / (a1 + a2s[pl.ds(base, R), :] - inter + 1e-8)
                snew = jnp.where(iou > _IOU_T, -jnp.inf, sv[c])
                ss[pl.ds(base, R), :] = snew
                new_ms.append(jnp.max(snew))
            return (step + 1,) + tuple(new_ms)

        jax.lax.while_loop(cond, body, (jnp.int32(0),) + m_init)

    @pl.when(maxt <= _FAST_ROWS * _LANES)
    def _():
        run(_FAST_ROWS)

    @pl.when(maxt > _FAST_ROWS * _LANES)
    def _():
        run(_SLOT_ROWS)


def _merge_kernel(ms, my1, mx1, my2, mx2, res, scr):
    crow = jax.lax.broadcasted_iota(jnp.int32, (_NUM_CLASSES, _LANES), 0)
    lane = jax.lax.broadcasted_iota(jnp.int32, (_NUM_CLASSES, _LANES), 1)
    lane1 = jax.lax.broadcasted_iota(jnp.int32, (1, _LANES), 1)
    validlane = lane < _MAX_DET_PER_CLASS
    flat = jnp.where(validlane, crow * _MAX_DET_PER_CLASS + lane,
                     jnp.int32(2 ** 30))

    scr[...] = jnp.where(validlane, ms[0], -jnp.inf)
    res[0] = jnp.zeros((_NUM_CLASSES, _LANES), jnp.float32)

    m0 = jnp.max(scr[...])

    def cond(carry):
        step, m = carry
        return (step < _MAX_DET) & (m > _CONF)

    def body(carry):
        step, m = carry
        sv = scr[...]
        eq = sv == m
        fidx = jnp.min(jnp.where(eq, flat, jnp.int32(2 ** 30)))
        c = fidx // _MAX_DET_PER_CLASS
        j = fidx % _MAX_DET_PER_CLASS
        mask = (crow == c) & (lane == j)
        by1 = jnp.sum(jnp.where(mask, my1[0], 0.0))
        bx1 = jnp.sum(jnp.where(mask, mx1[0], 0.0))
        by2 = jnp.sum(jnp.where(mask, my2[0], 0.0))
        bx2 = jnp.sum(jnp.where(mask, mx2[0], 0.0))

        lm = lane1 == step
        res[0, pl.ds(0, 1), :] = jnp.where(lm, bx1 * 512.0, res[0, pl.ds(0, 1), :])
        res[0, pl.ds(1, 1), :] = jnp.where(lm, by1 * 512.0, res[0, pl.ds(1, 1), :])
        res[0, pl.ds(2, 1), :] = jnp.where(lm, bx2 * 512.0, res[0, pl.ds(2, 1), :])
        res[0, pl.ds(3, 1), :] = jnp.where(lm, by2 * 512.0, res[0, pl.ds(3, 1), :])
        res[0, pl.ds(4, 1), :] = jnp.where(lm, c.astype(jnp.float32), res[0, pl.ds(4, 1), :])
        res[0, pl.ds(5, 1), :] = jnp.where(lm, m, res[0, pl.ds(5, 1), :])

        snew = jnp.where(mask, -jnp.inf, sv)
        scr[...] = snew
        return step + 1, jnp.max(snew)

    nstep, _ = jax.lax.while_loop(cond, body, (jnp.int32(0), m0))
    res[0, pl.ds(6, 1), :] = jnp.where(lane1 == 0, nstep.astype(jnp.float32),
                                       res[0, pl.ds(6, 1), :])


def _nms_from_compact(X1, Y1, X2, Y2, SC, cf):
    B = X1.shape[0]
    pay_spec = pl.BlockSpec((1, _NUM_CLASSES * _SLOT_ROWS, _LANES),
                            lambda b: (b, 0, 0))
    cf_spec = pl.BlockSpec((1, 1, _LANES), lambda b: (b, 0, 0))
    out_spec = pl.BlockSpec((1, _NUM_CLASSES, _LANES), lambda b: (b, 0, 0))
    out_shape = jax.ShapeDtypeStruct((B, _NUM_CLASSES, _LANES), jnp.float32)
    big = (_NUM_CLASSES * _SLOT_ROWS, _LANES)

    sel_s, sel_y1, sel_x1, sel_y2, sel_x2 = pl.pallas_call(
        _nms_kernel,
        grid=(B,),
        in_specs=[pay_spec] * 5 + [cf_spec],
        out_specs=[out_spec] * 5,
        out_shape=[out_shape] * 5,
        scratch_shapes=[pltpu.VMEM(big, jnp.float32)] * 6,
        compiler_params=pltpu.CompilerParams(
            dimension_semantics=("parallel",)),
    )(X1, Y1, X2, Y2, SC, cf)

    mspec = pl.BlockSpec((1, _NUM_CLASSES, _LANES), lambda b: (b, 0, 0))
    res = pl.pallas_call(
        _merge_kernel,
        grid=(B,),
        in_specs=[mspec] * 5,
        out_specs=mspec,
        out_shape=jax.ShapeDtypeStruct((B, _NUM_CLASSES, _LANES), jnp.float32),
        scratch_shapes=[pltpu.VMEM((_NUM_CLASSES, _LANES), jnp.float32)],
    )(sel_s, sel_y1, sel_x1, sel_y2, sel_x2)

    out6 = jnp.transpose(res[:, 0:6, 0:_MAX_DET], (0, 2, 1))
    valid_det = res[:, 6, 0].astype(jnp.int32)
    return out6, valid_det


@jax.jit
def kernel(images, predictions):
    B = predictions.shape[0]

    def _flat(a, pad_value):
        a = jnp.pad(a, ((0, 0), (0, _NPAD - _N)), constant_values=pad_value)
        return a.reshape(B * _NPAD)

    x1f = _flat(predictions[..., 0], 0.0)
    y1f = _flat(predictions[..., 1], 0.0)
    x2f = _flat(predictions[..., 2], 0.0)
    y2f = _flat(predictions[..., 3], 0.0)
    clsf = _flat(predictions[..., 4].astype(jnp.int32), _NUM_CLASSES)
    scf = _flat(predictions[..., 5], 0.0)

    ox1, oy1, ox2, oy2, osc, ocnt = _sc_compact(clsf, x1f, y1f, x2f, y2f, scf)

    shp = (B, _NUM_CLASSES * _SLOT_ROWS, _LANES)
    X1 = ox1.reshape(shp)
    Y1 = oy1.reshape(shp)
    X2 = ox2.reshape(shp)
    Y2 = oy2.reshape(shp)
    SCp = osc.reshape(shp)
    cnts = ocnt.reshape(B, _NUM_CLASSES, 16)[:, :, 0].astype(jnp.float32)
    cf = jnp.zeros((B, _LANES), jnp.float32).at[:, :_NUM_CLASSES].set(cnts)
    cf = cf.reshape(B, 1, _LANES)

    return _nms_from_compact(X1, Y1, X2, Y2, SCp, cf)


# speedup vs baseline: 1.4271x; 1.1792x over previous
import functools

import jax
import jax.numpy as jnp
from jax.experimental import pallas as pl
from jax.experimental.pallas import tpu as pltpu
from jax.experimental.pallas import tpu_sc as plsc

_NUM_CLASSES = 8
_CONF = 0.05
_IOU_T = 0.5
_MAX_DET = 100
_MAX_DET_PER_CLASS = 100

_N = 20000
_NPAD = 20480
_LANES = 128

_SLOT_ROWS = 168
_SLOT = _SLOT_ROWS * _LANES
_FAST_ROWS = 32
_CHUNK = 1280
_NCHUNKS = _NPAD // _CHUNK


def _compact_kernel(cls_hbm, x1_hbm, y1_hbm, x2_hbm, y2_hbm, sc_hbm,
                    ox1, oy1, ox2, oy2, osc, ocnt,
                    cls_v, px1, py1, px2, py2, psc,
                    cls_w, qx1, qy1, qx2, qy2, qsc,
                    st_x1, st_y1, st_x2, st_y2, st_sc, cnt_v,
                    sem_a, sem_b):
    b = jax.lax.axis_index("c")
    k = jax.lax.axis_index("s")

    @pl.when(k < _NUM_CLASSES)
    def _():
        myc = k

        def zbody(j, carry):
            st_sc[pl.ds(j * 16, 16)] = jnp.zeros((16,), jnp.float32)
            return carry

        jax.lax.fori_loop(0, (_SLOT + 16) // 16, zbody, jnp.int32(0))

        ones16 = jnp.full((16,), 1, jnp.int32)
        zeros16 = jnp.full((16,), 0, jnp.int32)
        lane16 = jnp.arange(16, dtype=jnp.int32)
        trash16 = lane16 + jnp.full((16,), _SLOT, jnp.int32)
        mycv = jnp.full((16,), myc, jnp.int32)
        sixteen16 = jnp.full((16,), 16, jnp.int32)

        bufs_a = (cls_v, px1, py1, px2, py2, psc)
        bufs_b = (cls_w, qx1, qy1, qx2, qy2, qsc)
        srcs = (cls_hbm, x1_hbm, y1_hbm, x2_hbm, y2_hbm, sc_hbm)

        def copies(t, bufs, sem):
            off = pl.multiple_of(b * _NPAD + t * _CHUNK, 8)
            return [pltpu.make_async_copy(s.at[pl.ds(off, _CHUNK)], d, sem)
                    for s, d in zip(srcs, bufs)]

        def issue(t, bufs, sem):
            for c in copies(t, bufs, sem):
                c.start()

        def wait(t, bufs, sem):
            for c in copies(t, bufs, sem):
                c.wait()

        def process(bufs, curv):
            cv, bx1, by1, bx2, by2, bsc = bufs

            def vec4(j, cur):
                for u in range(4):
                    i = j * 4 + u
                    v = cv[pl.ds(i * 16, 16)]
                    mask = v == mycv
                    pop = plsc.all_reduce_population_count(mask)
                    keys = jnp.where(mask, lane16, lane16 + sixteen16)
                    _, sval = plsc.sort_key_val(keys, lane16)
                    srcdst = jnp.where(lane16 < pop, cur + lane16, trash16)
                    plsc.store_scatter(cnt_v, [sval], srcdst)
                    dst = cnt_v[...]
                    plsc.store_scatter(st_x1, [dst], bx1[pl.ds(i * 16, 16)])
                    plsc.store_scatter(st_y1, [dst], by1[pl.ds(i * 16, 16)])
                    plsc.store_scatter(st_x2, [dst], bx2[pl.ds(i * 16, 16)])
                    plsc.store_scatter(st_y2, [dst], by2[pl.ds(i * 16, 16)])
                    plsc.store_scatter(st_sc, [dst], bsc[pl.ds(i * 16, 16)])
                    cur = cur + pop
                return cur

            return jax.lax.fori_loop(0, _CHUNK // 64, vec4, curv)

        issue(0, bufs_a, sem_a)

        def outer(t, curv):
            t2 = t * 2
            issue(t2 + 1, bufs_b, sem_b)
            wait(t2, bufs_a, sem_a)
            curv = process(bufs_a, curv)

            @pl.when(t < _NCHUNKS // 2 - 1)
            def _():
                issue(t2 + 2, bufs_a, sem_a)

            wait(t2 + 1, bufs_b, sem_b)
            curv = process(bufs_b, curv)
            return curv

        n_v = jax.lax.fori_loop(0, _NCHUNKS // 2, outer,
                                jnp.full((16,), 0, jnp.int32))
        npad_v = ((n_v + jnp.full((16,), 7, jnp.int32))
                  & jnp.full((16,), -8, jnp.int32))

        slot = pl.multiple_of((b * _NUM_CLASSES + myc) * _SLOT, 8)
        pltpu.sync_copy(st_x1.at[pl.ds(0, _SLOT)], ox1.at[pl.ds(slot, _SLOT)])
        pltpu.sync_copy(st_y1.at[pl.ds(0, _SLOT)], oy1.at[pl.ds(slot, _SLOT)])
        pltpu.sync_copy(st_x2.at[pl.ds(0, _SLOT)], ox2.at[pl.ds(slot, _SLOT)])
        pltpu.sync_copy(st_y2.at[pl.ds(0, _SLOT)], oy2.at[pl.ds(slot, _SLOT)])
        pltpu.sync_copy(st_sc.at[pl.ds(0, _SLOT)], osc.at[pl.ds(slot, _SLOT)])

        iota16 = jnp.arange(16, dtype=jnp.int32)
        cnt_v[...] = jnp.where(iota16 == 0, npad_v,
                               jnp.full((16,), 0, jnp.int32))
        coff = pl.multiple_of((b * _NUM_CLASSES + myc) * 16, 8)
        pltpu.sync_copy(cnt_v, ocnt.at[pl.ds(coff, 16)])


def _sc_compact(cls_f, x1f, y1f, x2f, y2f, scf):
    pay = jax.ShapeDtypeStruct((2 * _NUM_CLASSES * _SLOT,), jnp.float32)
    kfn = pl.kernel(
        _compact_kernel,
        mesh=plsc.VectorSubcoreMesh(core_axis_name="c", subcore_axis_name="s"),
        compiler_params=pltpu.CompilerParams(needs_layout_passes=False),
        out_type=[pay] * 5
        + [jax.ShapeDtypeStruct((2 * _NUM_CLASSES * 16,), jnp.int32)],
        scratch_types=([pltpu.VMEM((_CHUNK,), jnp.int32)]
                       + [pltpu.VMEM((_CHUNK,), jnp.float32)] * 5) * 2
        + [pltpu.VMEM((_SLOT + 16,), jnp.float32)] * 5
        + [pltpu.VMEM((16,), jnp.int32)]
        + [pltpu.SemaphoreType.DMA] * 2,
    )
    return kfn(cls_f, x1f, y1f, x2f, y2f, scf)


def _nms_kernel(x1, y1, x2, y2, sc, cf,
                sel_s, sel_y1, sel_x1, sel_y2, sel_x2,
                ny1, nx1, ny2, nx2, a2s, ss):
    NBC = 2 * _NUM_CLASSES
    lane = jax.lax.broadcasted_iota(jnp.int32, (1, _LANES), 1)
    tot = []
    for bc in range(NBC):
        b, c = divmod(bc, _NUM_CLASSES)
        tot.append(jnp.sum(jnp.where(lane == c, cf[b], 0.0))
                   .astype(jnp.int32))
    maxt = tot[0]
    for bc in range(1, NBC):
        maxt = jnp.maximum(maxt, tot[bc])

    neg = jnp.full((_NUM_CLASSES, _LANES), -jnp.inf, jnp.float32)
    zeros = jnp.zeros((_NUM_CLASSES, _LANES), jnp.float32)
    for b in range(2):
        sel_s[b] = neg
        sel_y1[b] = zeros
        sel_x1[b] = zeros
        sel_y2[b] = zeros
        sel_x2[b] = zeros

    def run(R):
        flat = (jax.lax.broadcasted_iota(jnp.int32, (R, _LANES), 0) * _LANES
                + jax.lax.broadcasted_iota(jnp.int32, (R, _LANES), 1))
        for bc in range(NBC):
            b, c = divmod(bc, _NUM_CLASSES)
            ib = c * _SLOT_ROWS
            base = bc * _SLOT_ROWS
            ny1v = y1[b, pl.ds(ib, R), :] / 512.0
            nx1v = x1[b, pl.ds(ib, R), :] / 512.0
            ny2v = y2[b, pl.ds(ib, R), :] / 512.0
            nx2v = x2[b, pl.ds(ib, R), :] / 512.0
            ny1[pl.ds(base, R), :] = ny1v
            nx1[pl.ds(base, R), :] = nx1v
            ny2[pl.ds(base, R), :] = ny2v
            nx2[pl.ds(base, R), :] = nx2v
            a2s[pl.ds(base, R), :] = (ny2v - ny1v) * (nx2v - nx1v)
            scv = sc[b, pl.ds(ib, R), :]
            ss[pl.ds(base, R), :] = jnp.where(
                (flat < tot[bc]) & (scv > _CONF), scv, -jnp.inf)

        m_init = tuple(jnp.max(ss[pl.ds(bc * _SLOT_ROWS, R), :])
                       for bc in range(NBC))

        def cond(carry):
            step = carry[0]
            ms = carry[1:]
            any_m = ms[0]
            for bc in range(1, NBC):
                any_m = jnp.maximum(any_m, ms[bc])
            return (step < _MAX_DET_PER_CLASS) & (any_m > _CONF)

        def body(carry):
            step = carry[0]
            ms = carry[1:]
            act = [ms[bc] > _CONF for bc in range(NBC)]
            sv = [ss[pl.ds(bc * _SLOT_ROWS, R), :] for bc in range(NBC)]
            idx = [None] * NBC
            for bc in range(NBC):
                eq = sv[bc] == ms[bc]
                idx[bc] = jnp.min(jnp.where(eq, flat, jnp.int32(2 ** 30)))
            row, colmask = [None] * NBC, [None] * NBC
            for bc in range(NBC):
                i = jnp.where(act[bc], idx[bc], 0)
                row[bc] = bc * _SLOT_ROWS + i // _LANES
                colmask[bc] = lane == (i % _LANES)
            by1 = [None] * NBC
            bx1 = [None] * NBC
            by2 = [None] * NBC
            bx2 = [None] * NBC
            for bc in range(NBC):
                by1[bc] = jnp.sum(jnp.where(colmask[bc], ny1[pl.ds(row[bc], 1), :], 0.0))
                bx1[bc] = jnp.sum(jnp.where(colmask[bc], nx1[pl.ds(row[bc], 1), :], 0.0))
                by2[bc] = jnp.sum(jnp.where(colmask[bc], ny2[pl.ds(row[bc], 1), :], 0.0))
                bx2[bc] = jnp.sum(jnp.where(colmask[bc], nx2[pl.ds(row[bc], 1), :], 0.0))
            for bc in range(NBC):
                b, c = divmod(bc, _NUM_CLASSES)
                lm = (lane == step) & act[bc]
                sel_s[b, pl.ds(c, 1), :] = jnp.where(lm, ms[bc], sel_s[b, pl.ds(c, 1), :])
                sel_y1[b, pl.ds(c, 1), :] = jnp.where(lm, by1[bc], sel_y1[b, pl.ds(c, 1), :])
                sel_x1[b, pl.ds(c, 1), :] = jnp.where(lm, bx1[bc], sel_x1[b, pl.ds(c, 1), :])
                sel_y2[b, pl.ds(c, 1), :] = jnp.where(lm, by2[bc], sel_y2[b, pl.ds(c, 1), :])
                sel_x2[b, pl.ds(c, 1), :] = jnp.where(lm, bx2[bc], sel_x2[b, pl.ds(c, 1), :])
            new_ms = []
            for bc in range(NBC):
                base = bc * _SLOT_ROWS
                yy1 = jnp.maximum(by1[bc], ny1[pl.ds(base, R), :])
                xx1 = jnp.maximum(bx1[bc], nx1[pl.ds(base, R), :])
                yy2 = jnp.minimum(by2[bc], ny2[pl.ds(base, R), :])
                xx2 = jnp.minimum(bx2[bc], nx2[pl.ds(base, R), :])
                inter = jnp.maximum(yy2 - yy1, 0.0) * jnp.maximum(xx2 - xx1, 0.0)
                a1 = (by2[bc] - by1[bc]) * (bx2[bc] - bx1[bc])
                iou = inter / (a1 + a2s[pl.ds(base, R), :] - inter + 1e-8)
                snew = jnp.where(iou > _IOU_T, -jnp.inf, sv[bc])
                ss[pl.ds(base, R), :] = snew
                new_ms.append(jnp.max(snew))
            return (step + 1,) + tuple(new_ms)

        jax.lax.while_loop(cond, body, (jnp.int32(0),) + m_init)

    @pl.when(maxt <= _FAST_ROWS * _LANES)
    def _():
        run(_FAST_ROWS)

    @pl.when(maxt > _FAST_ROWS * _LANES)
    def _():
        run(_SLOT_ROWS)


def _merge_kernel(ms, my1, mx1, my2, mx2, res, scr):
    crow = jax.lax.broadcasted_iota(jnp.int32, (_NUM_CLASSES, _LANES), 0)
    lane = jax.lax.broadcasted_iota(jnp.int32, (_NUM_CLASSES, _LANES), 1)
    lane1 = jax.lax.broadcasted_iota(jnp.int32, (1, _LANES), 1)
    validlane = lane < _MAX_DET_PER_CLASS
    flat = jnp.where(validlane, crow * _MAX_DET_PER_CLASS + lane,
                     jnp.int32(2 ** 30))

    scr[...] = jnp.where(validlane, ms[0], -jnp.inf)
    res[0] = jnp.zeros((_NUM_CLASSES, _LANES), jnp.float32)

    m0 = jnp.max(scr[...])

    def cond(carry):
        step, m = carry
        return (step < _MAX_DET) & (m > _CONF)

    def body(carry):
        step, m = carry
        sv = scr[...]
        eq = sv == m
        fidx = jnp.min(jnp.where(eq, flat, jnp.int32(2 ** 30)))
        c = fidx // _MAX_DET_PER_CLASS
        j = fidx % _MAX_DET_PER_CLASS
        mask = (crow == c) & (lane == j)
        by1 = jnp.sum(jnp.where(mask, my1[0], 0.0))
        bx1 = jnp.sum(jnp.where(mask, mx1[0], 0.0))
        by2 = jnp.sum(jnp.where(mask, my2[0], 0.0))
        bx2 = jnp.sum(jnp.where(mask, mx2[0], 0.0))

        lm = lane1 == step
        res[0, pl.ds(0, 1), :] = jnp.where(lm, bx1 * 512.0, res[0, pl.ds(0, 1), :])
        res[0, pl.ds(1, 1), :] = jnp.where(lm, by1 * 512.0, res[0, pl.ds(1, 1), :])
        res[0, pl.ds(2, 1), :] = jnp.where(lm, bx2 * 512.0, res[0, pl.ds(2, 1), :])
        res[0, pl.ds(3, 1), :] = jnp.where(lm, by2 * 512.0, res[0, pl.ds(3, 1), :])
        res[0, pl.ds(4, 1), :] = jnp.where(lm, c.astype(jnp.float32), res[0, pl.ds(4, 1), :])
        res[0, pl.ds(5, 1), :] = jnp.where(lm, m, res[0, pl.ds(5, 1), :])

        snew = jnp.where(mask, -jnp.inf, sv)
        scr[...] = snew
        return step + 1, jnp.max(snew)

    nstep, _ = jax.lax.while_loop(cond, body, (jnp.int32(0), m0))
    res[0, pl.ds(6, 1), :] = jnp.where(lane1 == 0, nstep.astype(jnp.float32),
                                       res[0, pl.ds(6, 1), :])


def _nms_from_compact(X1, Y1, X2, Y2, SC, cf):
    B = X1.shape[0]
    pay_spec = pl.BlockSpec((B, _NUM_CLASSES * _SLOT_ROWS, _LANES),
                            lambda i: (0, 0, 0))
    cf_spec = pl.BlockSpec((B, 1, _LANES), lambda i: (0, 0, 0))
    out_spec = pl.BlockSpec((B, _NUM_CLASSES, _LANES), lambda i: (0, 0, 0))
    out_shape = jax.ShapeDtypeStruct((B, _NUM_CLASSES, _LANES), jnp.float32)
    big = (B * _NUM_CLASSES * _SLOT_ROWS, _LANES)

    sel_s, sel_y1, sel_x1, sel_y2, sel_x2 = pl.pallas_call(
        _nms_kernel,
        grid=(1,),
        in_specs=[pay_spec] * 5 + [cf_spec],
        out_specs=[out_spec] * 5,
        out_shape=[out_shape] * 5,
        scratch_shapes=[pltpu.VMEM(big, jnp.float32)] * 6,
    )(X1, Y1, X2, Y2, SC, cf)

    mspec = pl.BlockSpec((1, _NUM_CLASSES, _LANES), lambda b: (b, 0, 0))
    res = pl.pallas_call(
        _merge_kernel,
        grid=(B,),
        in_specs=[mspec] * 5,
        out_specs=mspec,
        out_shape=jax.ShapeDtypeStruct((B, _NUM_CLASSES, _LANES), jnp.float32),
        scratch_shapes=[pltpu.VMEM((_NUM_CLASSES, _LANES), jnp.float32)],
    )(sel_s, sel_y1, sel_x1, sel_y2, sel_x2)

    out6 = jnp.transpose(res[:, 0:6, 0:_MAX_DET], (0, 2, 1))
    valid_det = res[:, 6, 0].astype(jnp.int32)
    return out6, valid_det


@jax.jit
def kernel(images, predictions):
    B = predictions.shape[0]

    def _flat(a, pad_value):
        a = jnp.pad(a, ((0, 0), (0, _NPAD - _N)), constant_values=pad_value)
        return a.reshape(B * _NPAD)

    x1f = _flat(predictions[..., 0], 0.0)
    y1f = _flat(predictions[..., 1], 0.0)
    x2f = _flat(predictions[..., 2], 0.0)
    y2f = _flat(predictions[..., 3], 0.0)
    clsf = _flat(predictions[..., 4].astype(jnp.int32), _NUM_CLASSES)
    scf = _flat(predictions[..., 5], 0.0)

    ox1, oy1, ox2, oy2, osc, ocnt = _sc_compact(clsf, x1f, y1f, x2f, y2f, scf)

    shp = (B, _NUM_CLASSES * _SLOT_ROWS, _LANES)
    X1 = ox1.reshape(shp)
    Y1 = oy1.reshape(shp)
    X2 = ox2.reshape(shp)
    Y2 = oy2.reshape(shp)
    SCp = osc.reshape(shp)
    cnts = ocnt.reshape(B, _NUM_CLASSES, 16)[:, :, 0].astype(jnp.float32)
    cf = jnp.zeros((B, _LANES), jnp.float32).at[:, :_NUM_CLASSES].set(cnts)
    cf = cf.reshape(B, 1, _LANES)

    return _nms_from_compact(X1, Y1, X2, Y2, SCp, cf)


# speedup vs baseline: 1.4493x; 1.0155x over previous
import functools

import jax
import jax.numpy as jnp
from jax.experimental import pallas as pl
from jax.experimental.pallas import tpu as pltpu
from jax.experimental.pallas import tpu_sc as plsc

_NUM_CLASSES = 8
_CONF = 0.05
_IOU_T = 0.5
_MAX_DET = 100
_MAX_DET_PER_CLASS = 100

_N = 20000
_NPAD = 20480
_LANES = 128

_SLOT_ROWS = 168
_SLOT = _SLOT_ROWS * _LANES
_FAST_ROWS = 24
_CHUNK = 1280
_NCHUNKS = _NPAD // _CHUNK


def _compact_kernel(cls_hbm, x1_hbm, y1_hbm, x2_hbm, y2_hbm, sc_hbm,
                    ox1, oy1, ox2, oy2, osc, ocnt,
                    cls_v, px1, py1, px2, py2, psc,
                    cls_w, qx1, qy1, qx2, qy2, qsc,
                    st_x1, st_y1, st_x2, st_y2, st_sc, cnt_v,
                    sem_a, sem_b):
    b = jax.lax.axis_index("c")
    k = jax.lax.axis_index("s")

    @pl.when(k < _NUM_CLASSES)
    def _():
        myc = k

        def zbody(j, carry):
            st_sc[pl.ds(j * 16, 16)] = jnp.zeros((16,), jnp.float32)
            return carry

        jax.lax.fori_loop(0, (_SLOT + 16) // 16, zbody, jnp.int32(0))

        ones16 = jnp.full((16,), 1, jnp.int32)
        zeros16 = jnp.full((16,), 0, jnp.int32)
        lane16 = jnp.arange(16, dtype=jnp.int32)
        trash16 = lane16 + jnp.full((16,), _SLOT, jnp.int32)
        mycv = jnp.full((16,), myc, jnp.int32)
        sixteen16 = jnp.full((16,), 16, jnp.int32)

        bufs_a = (cls_v, px1, py1, px2, py2, psc)
        bufs_b = (cls_w, qx1, qy1, qx2, qy2, qsc)
        srcs = (cls_hbm, x1_hbm, y1_hbm, x2_hbm, y2_hbm, sc_hbm)

        def copies(t, bufs, sem):
            off = pl.multiple_of(b * _NPAD + t * _CHUNK, 8)
            return [pltpu.make_async_copy(s.at[pl.ds(off, _CHUNK)], d, sem)
                    for s, d in zip(srcs, bufs)]

        def issue(t, bufs, sem):
            for c in copies(t, bufs, sem):
                c.start()

        def wait(t, bufs, sem):
            for c in copies(t, bufs, sem):
                c.wait()

        def process(bufs, curv):
            cv, bx1, by1, bx2, by2, bsc = bufs

            def vec4(j, cur):
                for u in range(4):
                    i = j * 4 + u
                    v = cv[pl.ds(i * 16, 16)]
                    mask = v == mycv
                    pop = plsc.all_reduce_population_count(mask)
                    keys = jnp.where(mask, lane16, lane16 + sixteen16)
                    _, sval = plsc.sort_key_val(keys, lane16)
                    srcdst = jnp.where(lane16 < pop, cur + lane16, trash16)
                    plsc.store_scatter(cnt_v, [sval], srcdst)
                    dst = cnt_v[...]
                    plsc.store_scatter(st_x1, [dst], bx1[pl.ds(i * 16, 16)])
                    plsc.store_scatter(st_y1, [dst], by1[pl.ds(i * 16, 16)])
                    plsc.store_scatter(st_x2, [dst], bx2[pl.ds(i * 16, 16)])
                    plsc.store_scatter(st_y2, [dst], by2[pl.ds(i * 16, 16)])
                    plsc.store_scatter(st_sc, [dst], bsc[pl.ds(i * 16, 16)])
                    cur = cur + pop
                return cur

            return jax.lax.fori_loop(0, _CHUNK // 64, vec4, curv)

        issue(0, bufs_a, sem_a)

        def outer(t, curv):
            t2 = t * 2
            issue(t2 + 1, bufs_b, sem_b)
            wait(t2, bufs_a, sem_a)
            curv = process(bufs_a, curv)

            @pl.when(t < _NCHUNKS // 2 - 1)
            def _():
                issue(t2 + 2, bufs_a, sem_a)

            wait(t2 + 1, bufs_b, sem_b)
            curv = process(bufs_b, curv)
            return curv

        n_v = jax.lax.fori_loop(0, _NCHUNKS // 2, outer,
                                jnp.full((16,), 0, jnp.int32))
        npad_v = ((n_v + jnp.full((16,), 7, jnp.int32))
                  & jnp.full((16,), -8, jnp.int32))

        slot = pl.multiple_of((b * _NUM_CLASSES + myc) * _SLOT, 8)
        pltpu.sync_copy(st_x1.at[pl.ds(0, _SLOT)], ox1.at[pl.ds(slot, _SLOT)])
        pltpu.sync_copy(st_y1.at[pl.ds(0, _SLOT)], oy1.at[pl.ds(slot, _SLOT)])
        pltpu.sync_copy(st_x2.at[pl.ds(0, _SLOT)], ox2.at[pl.ds(slot, _SLOT)])
        pltpu.sync_copy(st_y2.at[pl.ds(0, _SLOT)], oy2.at[pl.ds(slot, _SLOT)])
        pltpu.sync_copy(st_sc.at[pl.ds(0, _SLOT)], osc.at[pl.ds(slot, _SLOT)])

        iota16 = jnp.arange(16, dtype=jnp.int32)
        cnt_v[...] = jnp.where(iota16 == 0, npad_v,
                               jnp.full((16,), 0, jnp.int32))
        coff = pl.multiple_of((b * _NUM_CLASSES + myc) * 16, 8)
        pltpu.sync_copy(cnt_v, ocnt.at[pl.ds(coff, 16)])


def _sc_compact(cls_f, x1f, y1f, x2f, y2f, scf):
    pay = jax.ShapeDtypeStruct((2 * _NUM_CLASSES * _SLOT,), jnp.float32)
    kfn = pl.kernel(
        _compact_kernel,
        mesh=plsc.VectorSubcoreMesh(core_axis_name="c", subcore_axis_name="s"),
        compiler_params=pltpu.CompilerParams(needs_layout_passes=False),
        out_type=[pay] * 5
        + [jax.ShapeDtypeStruct((2 * _NUM_CLASSES * 16,), jnp.int32)],
        scratch_types=([pltpu.VMEM((_CHUNK,), jnp.int32)]
                       + [pltpu.VMEM((_CHUNK,), jnp.float32)] * 5) * 2
        + [pltpu.VMEM((_SLOT + 16,), jnp.float32)] * 5
        + [pltpu.VMEM((16,), jnp.int32)]
        + [pltpu.SemaphoreType.DMA] * 2,
    )
    return kfn(cls_f, x1f, y1f, x2f, y2f, scf)


def _nms_kernel(x1, y1, x2, y2, sc, cf,
                sel_s, sel_y1, sel_x1, sel_y2, sel_x2,
                ny1, nx1, ny2, nx2, a2s, ss):
    NBC = 2 * _NUM_CLASSES
    lane = jax.lax.broadcasted_iota(jnp.int32, (1, _LANES), 1)
    tot = []
    for bc in range(NBC):
        b, c = divmod(bc, _NUM_CLASSES)
        tot.append(jnp.sum(jnp.where(lane == c, cf[b], 0.0))
                   .astype(jnp.int32))
    maxt = tot[0]
    for bc in range(1, NBC):
        maxt = jnp.maximum(maxt, tot[bc])

    neg = jnp.full((_NUM_CLASSES, _LANES), -jnp.inf, jnp.float32)
    zeros = jnp.zeros((_NUM_CLASSES, _LANES), jnp.float32)
    for b in range(2):
        sel_s[b] = neg
        sel_y1[b] = zeros
        sel_x1[b] = zeros
        sel_y2[b] = zeros
        sel_x2[b] = zeros

    def run(R):
        flat = (jax.lax.broadcasted_iota(jnp.int32, (R, _LANES), 0) * _LANES
                + jax.lax.broadcasted_iota(jnp.int32, (R, _LANES), 1))
        for bc in range(NBC):
            b, c = divmod(bc, _NUM_CLASSES)
            ib = c * _SLOT_ROWS
            base = bc * _SLOT_ROWS
            ny1v = y1[b, pl.ds(ib, R), :] / 512.0
            nx1v = x1[b, pl.ds(ib, R), :] / 512.0
            ny2v = y2[b, pl.ds(ib, R), :] / 512.0
            nx2v = x2[b, pl.ds(ib, R), :] / 512.0
            ny1[pl.ds(base, R), :] = ny1v
            nx1[pl.ds(base, R), :] = nx1v
            ny2[pl.ds(base, R), :] = ny2v
            nx2[pl.ds(base, R), :] = nx2v
            a2s[pl.ds(base, R), :] = (ny2v - ny1v) * (nx2v - nx1v)
            scv = sc[b, pl.ds(ib, R), :]
            ss[pl.ds(base, R), :] = jnp.where(
                (flat < tot[bc]) & (scv > _CONF), scv, -jnp.inf)

        m_init = tuple(jnp.max(ss[pl.ds(bc * _SLOT_ROWS, R), :])
                       for bc in range(NBC))

        def cond(carry):
            step = carry[0]
            ms = carry[1:]
            any_m = ms[0]
            for bc in range(1, NBC):
                any_m = jnp.maximum(any_m, ms[bc])
            return (step < _MAX_DET_PER_CLASS) & (any_m > _CONF)

        def body(carry):
            step = carry[0]
            ms = carry[1:]
            act = [ms[bc] > _CONF for bc in range(NBC)]
            sv = [ss[pl.ds(bc * _SLOT_ROWS, R), :] for bc in range(NBC)]
            idx = [None] * NBC
            for bc in range(NBC):
                eq = sv[bc] == ms[bc]
                idx[bc] = jnp.min(jnp.where(eq, flat, jnp.int32(2 ** 30)))
            row, colmask = [None] * NBC, [None] * NBC
            for bc in range(NBC):
                i = jnp.where(act[bc], idx[bc], 0)
                row[bc] = bc * _SLOT_ROWS + i // _LANES
                colmask[bc] = lane == (i % _LANES)
            by1 = [None] * NBC
            bx1 = [None] * NBC
            by2 = [None] * NBC
            bx2 = [None] * NBC
            for bc in range(NBC):
                by1[bc] = jnp.sum(jnp.where(colmask[bc], ny1[pl.ds(row[bc], 1), :], 0.0))
                bx1[bc] = jnp.sum(jnp.where(colmask[bc], nx1[pl.ds(row[bc], 1), :], 0.0))
                by2[bc] = jnp.sum(jnp.where(colmask[bc], ny2[pl.ds(row[bc], 1), :], 0.0))
                bx2[bc] = jnp.sum(jnp.where(colmask[bc], nx2[pl.ds(row[bc], 1), :], 0.0))
            for bc in range(NBC):
                b, c = divmod(bc, _NUM_CLASSES)
                lm = (lane == step) & act[bc]
                sel_s[b, pl.ds(c, 1), :] = jnp.where(lm, ms[bc], sel_s[b, pl.ds(c, 1), :])
                sel_y1[b, pl.ds(c, 1), :] = jnp.where(lm, by1[bc], sel_y1[b, pl.ds(c, 1), :])
                sel_x1[b, pl.ds(c, 1), :] = jnp.where(lm, bx1[bc], sel_x1[b, pl.ds(c, 1), :])
                sel_y2[b, pl.ds(c, 1), :] = jnp.where(lm, by2[bc], sel_y2[b, pl.ds(c, 1), :])
                sel_x2[b, pl.ds(c, 1), :] = jnp.where(lm, bx2[bc], sel_x2[b, pl.ds(c, 1), :])
            new_ms = []
            for bc in range(NBC):
                base = bc * _SLOT_ROWS
                yy1 = jnp.maximum(by1[bc], ny1[pl.ds(base, R), :])
                xx1 = jnp.maximum(bx1[bc], nx1[pl.ds(base, R), :])
                yy2 = jnp.minimum(by2[bc], ny2[pl.ds(base, R), :])
                xx2 = jnp.minimum(bx2[bc], nx2[pl.ds(base, R), :])
                inter = jnp.maximum(yy2 - yy1, 0.0) * jnp.maximum(xx2 - xx1, 0.0)
                a1 = (by2[bc] - by1[bc]) * (bx2[bc] - bx1[bc])
                iou = inter / (a1 + a2s[pl.ds(base, R), :] - inter + 1e-8)
                snew = jnp.where(iou > _IOU_T, -jnp.inf, sv[bc])
                ss[pl.ds(base, R), :] = snew
                new_ms.append(jnp.max(snew))
            return (step + 1,) + tuple(new_ms)

        jax.lax.while_loop(cond, body, (jnp.int32(0),) + m_init)

    @pl.when(maxt <= _FAST_ROWS * _LANES)
    def _():
        run(_FAST_ROWS)

    @pl.when(maxt > _FAST_ROWS * _LANES)
    def _():
        run(_SLOT_ROWS)


def _merge_kernel(ms, my1, mx1, my2, mx2, res, scr):
    crow = jax.lax.broadcasted_iota(jnp.int32, (_NUM_CLASSES, _LANES), 0)
    lane = jax.lax.broadcasted_iota(jnp.int32, (_NUM_CLASSES, _LANES), 1)
    lane1 = jax.lax.broadcasted_iota(jnp.int32, (1, _LANES), 1)
    validlane = lane < _MAX_DET_PER_CLASS
    flat = jnp.where(validlane, crow * _MAX_DET_PER_CLASS + lane,
                     jnp.int32(2 ** 30))

    scr[...] = jnp.where(validlane, ms[0], -jnp.inf)
    res[0] = jnp.zeros((_NUM_CLASSES, _LANES), jnp.float32)

    m0 = jnp.max(scr[...])

    def cond(carry):
        step, m = carry
        return (step < _MAX_DET) & (m > _CONF)

    def body(carry):
        step, m = carry
        sv = scr[...]
        eq = sv == m
        fidx = jnp.min(jnp.where(eq, flat, jnp.int32(2 ** 30)))
        c = fidx // _MAX_DET_PER_CLASS
        j = fidx % _MAX_DET_PER_CLASS
        mask = (crow == c) & (lane == j)
        by1 = jnp.sum(jnp.where(mask, my1[0], 0.0))
        bx1 = jnp.sum(jnp.where(mask, mx1[0], 0.0))
        by2 = jnp.sum(jnp.where(mask, my2[0], 0.0))
        bx2 = jnp.sum(jnp.where(mask, mx2[0], 0.0))

        lm = lane1 == step
        res[0, pl.ds(0, 1), :] = jnp.where(lm, bx1 * 512.0, res[0, pl.ds(0, 1), :])
        res[0, pl.ds(1, 1), :] = jnp.where(lm, by1 * 512.0, res[0, pl.ds(1, 1), :])
        res[0, pl.ds(2, 1), :] = jnp.where(lm, bx2 * 512.0, res[0, pl.ds(2, 1), :])
        res[0, pl.ds(3, 1), :] = jnp.where(lm, by2 * 512.0, res[0, pl.ds(3, 1), :])
        res[0, pl.ds(4, 1), :] = jnp.where(lm, c.astype(jnp.float32), res[0, pl.ds(4, 1), :])
        res[0, pl.ds(5, 1), :] = jnp.where(lm, m, res[0, pl.ds(5, 1), :])

        snew = jnp.where(mask, -jnp.inf, sv)
        scr[...] = snew
        return step + 1, jnp.max(snew)

    nstep, _ = jax.lax.while_loop(cond, body, (jnp.int32(0), m0))
    res[0, pl.ds(6, 1), :] = jnp.where(lane1 == 0, nstep.astype(jnp.float32),
                                       res[0, pl.ds(6, 1), :])


def _nms_from_compact(X1, Y1, X2, Y2, SC, cf):
    B = X1.shape[0]
    pay_spec = pl.BlockSpec((B, _NUM_CLASSES * _SLOT_ROWS, _LANES),
                            lambda i: (0, 0, 0))
    cf_spec = pl.BlockSpec((B, 1, _LANES), lambda i: (0, 0, 0))
    out_spec = pl.BlockSpec((B, _NUM_CLASSES, _LANES), lambda i: (0, 0, 0))
    out_shape = jax.ShapeDtypeStruct((B, _NUM_CLASSES, _LANES), jnp.float32)
    big = (B * _NUM_CLASSES * _SLOT_ROWS, _LANES)

    sel_s, sel_y1, sel_x1, sel_y2, sel_x2 = pl.pallas_call(
        _nms_kernel,
        grid=(1,),
        in_specs=[pay_spec] * 5 + [cf_spec],
        out_specs=[out_spec] * 5,
        out_shape=[out_shape] * 5,
        scratch_shapes=[pltpu.VMEM(big, jnp.float32)] * 6,
    )(X1, Y1, X2, Y2, SC, cf)

    mspec = pl.BlockSpec((1, _NUM_CLASSES, _LANES), lambda b: (b, 0, 0))
    res = pl.pallas_call(
        _merge_kernel,
        grid=(B,),
        in_specs=[mspec] * 5,
        out_specs=mspec,
        out_shape=jax.ShapeDtypeStruct((B, _NUM_CLASSES, _LANES), jnp.float32),
        scratch_shapes=[pltpu.VMEM((_NUM_CLASSES, _LANES), jnp.float32)],
    )(sel_s, sel_y1, sel_x1, sel_y2, sel_x2)

    out6 = jnp.transpose(res[:, 0:6, 0:_MAX_DET], (0, 2, 1))
    valid_det = res[:, 6, 0].astype(jnp.int32)
    return out6, valid_det


@jax.jit
def kernel(images, predictions):
    B = predictions.shape[0]

    def _flat(a, pad_value):
        a = jnp.pad(a, ((0, 0), (0, _NPAD - _N)), constant_values=pad_value)
        return a.reshape(B * _NPAD)

    x1f = _flat(predictions[..., 0], 0.0)
    y1f = _flat(predictions[..., 1], 0.0)
    x2f = _flat(predictions[..., 2], 0.0)
    y2f = _flat(predictions[..., 3], 0.0)
    clsf = _flat(predictions[..., 4].astype(jnp.int32), _NUM_CLASSES)
    scf = _flat(predictions[..., 5], 0.0)

    ox1, oy1, ox2, oy2, osc, ocnt = _sc_compact(clsf, x1f, y1f, x2f, y2f, scf)

    shp = (B, _NUM_CLASSES * _SLOT_ROWS, _LANES)
    X1 = ox1.reshape(shp)
    Y1 = oy1.reshape(shp)
    X2 = ox2.reshape(shp)
    Y2 = oy2.reshape(shp)
    SCp = osc.reshape(shp)
    cnts = ocnt.reshape(B, _NUM_CLASSES, 16)[:, :, 0].astype(jnp.float32)
    cf = jnp.zeros((B, _LANES), jnp.float32).at[:, :_NUM_CLASSES].set(cnts)
    cf = cf.reshape(B, 1, _LANES)

    return _nms_from_compact(X1, Y1, X2, Y2, SCp, cf)


# speedup vs baseline: 1.6026x; 1.1058x over previous
import functools

import jax
import jax.numpy as jnp
from jax.experimental import pallas as pl
from jax.experimental.pallas import tpu as pltpu
from jax.experimental.pallas import tpu_sc as plsc

_NUM_CLASSES = 8
_CONF = 0.05
_IOU_T = 0.5
_MAX_DET = 100
_MAX_DET_PER_CLASS = 100

_N = 20000
_NPAD = 20480
_LANES = 128

_SLOT_ROWS = 168
_SLOT = _SLOT_ROWS * _LANES
_FAST_ROWS = 24
_CHUNK = 1280
_NCHUNKS = _NPAD // _CHUNK


def _compact_kernel(cls_hbm, x1_hbm, y1_hbm, x2_hbm, y2_hbm, sc_hbm,
                    ox1, oy1, ox2, oy2, osc, ocnt,
                    cls_v, px1, py1, px2, py2, psc,
                    cls_w, qx1, qy1, qx2, qy2, qsc,
                    st_x1, st_y1, st_x2, st_y2, st_sc, cnt_v,
                    sem_a, sem_b):
    b = jax.lax.axis_index("c")
    k = jax.lax.axis_index("s")

    @pl.when(k < _NUM_CLASSES)
    def _():
        myc = k

        def zbody(j, carry):
            st_sc[pl.ds(j * 16, 16)] = jnp.zeros((16,), jnp.float32)
            return carry

        jax.lax.fori_loop(0, (_SLOT + 16) // 16, zbody, jnp.int32(0))

        ones16 = jnp.full((16,), 1, jnp.int32)
        zeros16 = jnp.full((16,), 0, jnp.int32)
        lane16 = jnp.arange(16, dtype=jnp.int32)
        trash16 = lane16 + jnp.full((16,), _SLOT, jnp.int32)
        mycv = jnp.full((16,), myc, jnp.int32)
        sixteen16 = jnp.full((16,), 16, jnp.int32)

        bufs_a = (cls_v, px1, py1, px2, py2, psc)
        bufs_b = (cls_w, qx1, qy1, qx2, qy2, qsc)
        srcs = (cls_hbm, x1_hbm, y1_hbm, x2_hbm, y2_hbm, sc_hbm)

        def copies(t, bufs, sem):
            off = pl.multiple_of(b * _NPAD + t * _CHUNK, 8)
            return [pltpu.make_async_copy(s.at[pl.ds(off, _CHUNK)], d, sem)
                    for s, d in zip(srcs, bufs)]

        def issue(t, bufs, sem):
            for c in copies(t, bufs, sem):
                c.start()

        def wait(t, bufs, sem):
            for c in copies(t, bufs, sem):
                c.wait()

        def process(bufs, curv):
            cv, bx1, by1, bx2, by2, bsc = bufs

            def vec4(j, cur):
                for u in range(4):
                    i = j * 4 + u
                    v = cv[pl.ds(i * 16, 16)]
                    mask = v == mycv
                    pop = plsc.all_reduce_population_count(mask)
                    keys = jnp.where(mask, lane16, lane16 + sixteen16)
                    _, sval = plsc.sort_key_val(keys, lane16)
                    srcdst = jnp.where(lane16 < pop, cur + lane16, trash16)
                    plsc.store_scatter(cnt_v, [sval], srcdst)
                    dst = cnt_v[...]
                    plsc.store_scatter(st_x1, [dst], bx1[pl.ds(i * 16, 16)])
                    plsc.store_scatter(st_y1, [dst], by1[pl.ds(i * 16, 16)])
                    plsc.store_scatter(st_x2, [dst], bx2[pl.ds(i * 16, 16)])
                    plsc.store_scatter(st_y2, [dst], by2[pl.ds(i * 16, 16)])
                    plsc.store_scatter(st_sc, [dst], bsc[pl.ds(i * 16, 16)])
                    cur = cur + pop
                return cur

            return jax.lax.fori_loop(0, _CHUNK // 64, vec4, curv)

        issue(0, bufs_a, sem_a)

        def outer(t, curv):
            t2 = t * 2
            issue(t2 + 1, bufs_b, sem_b)
            wait(t2, bufs_a, sem_a)
            curv = process(bufs_a, curv)

            @pl.when(t < _NCHUNKS // 2 - 1)
            def _():
                issue(t2 + 2, bufs_a, sem_a)

            wait(t2 + 1, bufs_b, sem_b)
            curv = process(bufs_b, curv)
            return curv

        n_v = jax.lax.fori_loop(0, _NCHUNKS // 2, outer,
                                jnp.full((16,), 0, jnp.int32))
        npad_v = ((n_v + jnp.full((16,), 7, jnp.int32))
                  & jnp.full((16,), -8, jnp.int32))

        slot = pl.multiple_of((b * _NUM_CLASSES + myc) * _SLOT, 8)
        pltpu.sync_copy(st_x1.at[pl.ds(0, _SLOT)], ox1.at[pl.ds(slot, _SLOT)])
        pltpu.sync_copy(st_y1.at[pl.ds(0, _SLOT)], oy1.at[pl.ds(slot, _SLOT)])
        pltpu.sync_copy(st_x2.at[pl.ds(0, _SLOT)], ox2.at[pl.ds(slot, _SLOT)])
        pltpu.sync_copy(st_y2.at[pl.ds(0, _SLOT)], oy2.at[pl.ds(slot, _SLOT)])
        pltpu.sync_copy(st_sc.at[pl.ds(0, _SLOT)], osc.at[pl.ds(slot, _SLOT)])

        iota16 = jnp.arange(16, dtype=jnp.int32)
        cnt_v[...] = jnp.where(iota16 == 0, npad_v,
                               jnp.full((16,), 0, jnp.int32))
        coff = pl.multiple_of((b * _NUM_CLASSES + myc) * 16, 8)
        pltpu.sync_copy(cnt_v, ocnt.at[pl.ds(coff, 16)])


def _sc_compact(cls_f, x1f, y1f, x2f, y2f, scf):
    pay = jax.ShapeDtypeStruct((2 * _NUM_CLASSES * _SLOT,), jnp.float32)
    kfn = pl.kernel(
        _compact_kernel,
        mesh=plsc.VectorSubcoreMesh(core_axis_name="c", subcore_axis_name="s"),
        compiler_params=pltpu.CompilerParams(needs_layout_passes=False),
        out_type=[pay] * 5
        + [jax.ShapeDtypeStruct((2 * _NUM_CLASSES * 16,), jnp.int32)],
        scratch_types=([pltpu.VMEM((_CHUNK,), jnp.int32)]
                       + [pltpu.VMEM((_CHUNK,), jnp.float32)] * 5) * 2
        + [pltpu.VMEM((_SLOT + 16,), jnp.float32)] * 5
        + [pltpu.VMEM((16,), jnp.int32)]
        + [pltpu.SemaphoreType.DMA] * 2,
    )
    return kfn(cls_f, x1f, y1f, x2f, y2f, scf)


def _nms_kernel(x1, y1, x2, y2, sc, cf,
                sel_s, sel_y1, sel_x1, sel_y2, sel_x2, res,
                ny1, nx1, ny2, nx2, a2s, ss, mscr):
    NBC = 2 * _NUM_CLASSES
    lane = jax.lax.broadcasted_iota(jnp.int32, (1, _LANES), 1)
    tot = []
    for bc in range(NBC):
        b, c = divmod(bc, _NUM_CLASSES)
        tot.append(jnp.sum(jnp.where(lane == c, cf[b], 0.0))
                   .astype(jnp.int32))
    maxt = tot[0]
    for bc in range(1, NBC):
        maxt = jnp.maximum(maxt, tot[bc])

    neg = jnp.full((_NUM_CLASSES, _LANES), -jnp.inf, jnp.float32)
    zeros = jnp.zeros((_NUM_CLASSES, _LANES), jnp.float32)
    for b in range(2):
        sel_s[b] = neg
        sel_y1[b] = zeros
        sel_x1[b] = zeros
        sel_y2[b] = zeros
        sel_x2[b] = zeros

    def run(R):
        flat = (jax.lax.broadcasted_iota(jnp.int32, (R, _LANES), 0) * _LANES
                + jax.lax.broadcasted_iota(jnp.int32, (R, _LANES), 1))
        for bc in range(NBC):
            b, c = divmod(bc, _NUM_CLASSES)
            ib = c * _SLOT_ROWS
            base = bc * _SLOT_ROWS
            ny1v = y1[b, pl.ds(ib, R), :] / 512.0
            nx1v = x1[b, pl.ds(ib, R), :] / 512.0
            ny2v = y2[b, pl.ds(ib, R), :] / 512.0
            nx2v = x2[b, pl.ds(ib, R), :] / 512.0
            ny1[pl.ds(base, R), :] = ny1v
            nx1[pl.ds(base, R), :] = nx1v
            ny2[pl.ds(base, R), :] = ny2v
            nx2[pl.ds(base, R), :] = nx2v
            a2s[pl.ds(base, R), :] = (ny2v - ny1v) * (nx2v - nx1v)
            scv = sc[b, pl.ds(ib, R), :]
            ss[pl.ds(base, R), :] = jnp.where(
                (flat < tot[bc]) & (scv > _CONF), scv, -jnp.inf)

        m_init = tuple(jnp.max(ss[pl.ds(bc * _SLOT_ROWS, R), :])
                       for bc in range(NBC))

        def cond(carry):
            step = carry[0]
            ms = carry[1:]
            any_m = ms[0]
            for bc in range(1, NBC):
                any_m = jnp.maximum(any_m, ms[bc])
            return (step < _MAX_DET_PER_CLASS) & (any_m > _CONF)

        def body(carry):
            step = carry[0]
            ms = carry[1:]
            act = [ms[bc] > _CONF for bc in range(NBC)]
            sv = [ss[pl.ds(bc * _SLOT_ROWS, R), :] for bc in range(NBC)]
            idx = [None] * NBC
            for bc in range(NBC):
                eq = sv[bc] == ms[bc]
                idx[bc] = jnp.min(jnp.where(eq, flat, jnp.int32(2 ** 30)))
            row, colmask = [None] * NBC, [None] * NBC
            for bc in range(NBC):
                i = jnp.where(act[bc], idx[bc], 0)
                row[bc] = bc * _SLOT_ROWS + i // _LANES
                colmask[bc] = lane == (i % _LANES)
            by1 = [None] * NBC
            bx1 = [None] * NBC
            by2 = [None] * NBC
            bx2 = [None] * NBC
            for bc in range(NBC):
                by1[bc] = jnp.sum(jnp.where(colmask[bc], ny1[pl.ds(row[bc], 1), :], 0.0))
                bx1[bc] = jnp.sum(jnp.where(colmask[bc], nx1[pl.ds(row[bc], 1), :], 0.0))
                by2[bc] = jnp.sum(jnp.where(colmask[bc], ny2[pl.ds(row[bc], 1), :], 0.0))
                bx2[bc] = jnp.sum(jnp.where(colmask[bc], nx2[pl.ds(row[bc], 1), :], 0.0))
            for bc in range(NBC):
                b, c = divmod(bc, _NUM_CLASSES)
                lm = (lane == step) & act[bc]
                sel_s[b, pl.ds(c, 1), :] = jnp.where(lm, ms[bc], sel_s[b, pl.ds(c, 1), :])
                sel_y1[b, pl.ds(c, 1), :] = jnp.where(lm, by1[bc], sel_y1[b, pl.ds(c, 1), :])
                sel_x1[b, pl.ds(c, 1), :] = jnp.where(lm, bx1[bc], sel_x1[b, pl.ds(c, 1), :])
                sel_y2[b, pl.ds(c, 1), :] = jnp.where(lm, by2[bc], sel_y2[b, pl.ds(c, 1), :])
                sel_x2[b, pl.ds(c, 1), :] = jnp.where(lm, bx2[bc], sel_x2[b, pl.ds(c, 1), :])
            new_ms = []
            for bc in range(NBC):
                base = bc * _SLOT_ROWS
                yy1 = jnp.maximum(by1[bc], ny1[pl.ds(base, R), :])
                xx1 = jnp.maximum(bx1[bc], nx1[pl.ds(base, R), :])
                yy2 = jnp.minimum(by2[bc], ny2[pl.ds(base, R), :])
                xx2 = jnp.minimum(bx2[bc], nx2[pl.ds(base, R), :])
                inter = jnp.maximum(yy2 - yy1, 0.0) * jnp.maximum(xx2 - xx1, 0.0)
                a1 = (by2[bc] - by1[bc]) * (bx2[bc] - bx1[bc])
                iou = inter / (a1 + a2s[pl.ds(base, R), :] - inter + 1e-8)
                snew = jnp.where(iou > _IOU_T, -jnp.inf, sv[bc])
                ss[pl.ds(base, R), :] = snew
                new_ms.append(jnp.max(snew))
            return (step + 1,) + tuple(new_ms)

        jax.lax.while_loop(cond, body, (jnp.int32(0),) + m_init)

    @pl.when(maxt <= _FAST_ROWS * _LANES)
    def _():
        run(_FAST_ROWS)

    @pl.when(maxt > _FAST_ROWS * _LANES)
    def _():
        run(_SLOT_ROWS)

    crow = jax.lax.broadcasted_iota(jnp.int32, (_NUM_CLASSES, _LANES), 0)
    lane8 = jax.lax.broadcasted_iota(jnp.int32, (_NUM_CLASSES, _LANES), 1)
    validlane = lane8 < _MAX_DET_PER_CLASS
    flat8 = jnp.where(validlane, crow * _MAX_DET_PER_CLASS + lane8,
                      jnp.int32(2 ** 30))

    for b in range(2):
        mscr[pl.ds(b * _NUM_CLASSES, _NUM_CLASSES), :] = jnp.where(
            validlane, sel_s[b], -jnp.inf)
        res[b] = jnp.zeros((_NUM_CLASSES, _LANES), jnp.float32)

    mm = tuple(jnp.max(mscr[pl.ds(b * _NUM_CLASSES, _NUM_CLASSES), :])
               for b in range(2))

    def mcond(carry):
        step = carry[0]
        return (step < _MAX_DET) & (jnp.maximum(carry[1], carry[2]) > _CONF)

    def mbody(carry):
        step, ma0, ma1, cn0, cn1 = carry
        ms2 = (ma0, ma1)
        cns = [cn0, cn1]
        new_ms = []
        for b in range(2):
            m = ms2[b]
            act = m > _CONF
            sv = mscr[pl.ds(b * _NUM_CLASSES, _NUM_CLASSES), :]
            eq = sv == m
            fidx = jnp.min(jnp.where(eq, flat8, jnp.int32(2 ** 30)))
            fidx = jnp.where(act, fidx, 0)
            c = fidx // _MAX_DET_PER_CLASS
            j = fidx % _MAX_DET_PER_CLASS
            mask = (crow == c) & (lane8 == j)
            by1 = jnp.sum(jnp.where(mask, sel_y1[b], 0.0))
            bx1 = jnp.sum(jnp.where(mask, sel_x1[b], 0.0))
            by2 = jnp.sum(jnp.where(mask, sel_y2[b], 0.0))
            bx2 = jnp.sum(jnp.where(mask, sel_x2[b], 0.0))

            lm = (lane == step) & act
            res[b, pl.ds(0, 1), :] = jnp.where(lm, bx1 * 512.0, res[b, pl.ds(0, 1), :])
            res[b, pl.ds(1, 1), :] = jnp.where(lm, by1 * 512.0, res[b, pl.ds(1, 1), :])
            res[b, pl.ds(2, 1), :] = jnp.where(lm, bx2 * 512.0, res[b, pl.ds(2, 1), :])
            res[b, pl.ds(3, 1), :] = jnp.where(lm, by2 * 512.0, res[b, pl.ds(3, 1), :])
            res[b, pl.ds(4, 1), :] = jnp.where(lm, c.astype(jnp.float32), res[b, pl.ds(4, 1), :])
            res[b, pl.ds(5, 1), :] = jnp.where(lm, m, res[b, pl.ds(5, 1), :])

            snew = jnp.where(mask & act, -jnp.inf, sv)
            mscr[pl.ds(b * _NUM_CLASSES, _NUM_CLASSES), :] = snew
            new_ms.append(jnp.max(snew))
            cns[b] = cns[b] + jnp.where(act, jnp.int32(1), jnp.int32(0))
        return (step + 1, new_ms[0], new_ms[1], cns[0], cns[1])

    fin = jax.lax.while_loop(
        mcond, mbody, (jnp.int32(0), mm[0], mm[1], jnp.int32(0), jnp.int32(0)))
    for b in range(2):
        res[b, pl.ds(6, 1), :] = jnp.where(lane == 0,
                                           fin[3 + b].astype(jnp.float32),
                                           res[b, pl.ds(6, 1), :])


def _merge_kernel(ms, my1, mx1, my2, mx2, res, scr):
    crow = jax.lax.broadcasted_iota(jnp.int32, (_NUM_CLASSES, _LANES), 0)
    lane = jax.lax.broadcasted_iota(jnp.int32, (_NUM_CLASSES, _LANES), 1)
    lane1 = jax.lax.broadcasted_iota(jnp.int32, (1, _LANES), 1)
    validlane = lane < _MAX_DET_PER_CLASS
    flat = jnp.where(validlane, crow * _MAX_DET_PER_CLASS + lane,
                     jnp.int32(2 ** 30))

    scr[...] = jnp.where(validlane, ms[0], -jnp.inf)
    res[0] = jnp.zeros((_NUM_CLASSES, _LANES), jnp.float32)

    m0 = jnp.max(scr[...])

    def cond(carry):
        step, m = carry
        return (step < _MAX_DET) & (m > _CONF)

    def body(carry):
        step, m = carry
        sv = scr[...]
        eq = sv == m
        fidx = jnp.min(jnp.where(eq, flat, jnp.int32(2 ** 30)))
        c = fidx // _MAX_DET_PER_CLASS
        j = fidx % _MAX_DET_PER_CLASS
        mask = (crow == c) & (lane == j)
        by1 = jnp.sum(jnp.where(mask, my1[0], 0.0))
        bx1 = jnp.sum(jnp.where(mask, mx1[0], 0.0))
        by2 = jnp.sum(jnp.where(mask, my2[0], 0.0))
        bx2 = jnp.sum(jnp.where(mask, mx2[0], 0.0))

        lm = lane1 == step
        res[0, pl.ds(0, 1), :] = jnp.where(lm, bx1 * 512.0, res[0, pl.ds(0, 1), :])
        res[0, pl.ds(1, 1), :] = jnp.where(lm, by1 * 512.0, res[0, pl.ds(1, 1), :])
        res[0, pl.ds(2, 1), :] = jnp.where(lm, bx2 * 512.0, res[0, pl.ds(2, 1), :])
        res[0, pl.ds(3, 1), :] = jnp.where(lm, by2 * 512.0, res[0, pl.ds(3, 1), :])
        res[0, pl.ds(4, 1), :] = jnp.where(lm, c.astype(jnp.float32), res[0, pl.ds(4, 1), :])
        res[0, pl.ds(5, 1), :] = jnp.where(lm, m, res[0, pl.ds(5, 1), :])

        snew = jnp.where(mask, -jnp.inf, sv)
        scr[...] = snew
        return step + 1, jnp.max(snew)

    nstep, _ = jax.lax.while_loop(cond, body, (jnp.int32(0), m0))
    res[0, pl.ds(6, 1), :] = jnp.where(lane1 == 0, nstep.astype(jnp.float32),
                                       res[0, pl.ds(6, 1), :])


def _nms_from_compact(X1, Y1, X2, Y2, SC, cf):
    B = X1.shape[0]
    pay_spec = pl.BlockSpec((B, _NUM_CLASSES * _SLOT_ROWS, _LANES),
                            lambda i: (0, 0, 0))
    cf_spec = pl.BlockSpec((B, 1, _LANES), lambda i: (0, 0, 0))
    out_spec = pl.BlockSpec((B, _NUM_CLASSES, _LANES), lambda i: (0, 0, 0))
    out_shape = jax.ShapeDtypeStruct((B, _NUM_CLASSES, _LANES), jnp.float32)
    big = (B * _NUM_CLASSES * _SLOT_ROWS, _LANES)

    outs = pl.pallas_call(
        _nms_kernel,
        grid=(1,),
        in_specs=[pay_spec] * 5 + [cf_spec],
        out_specs=[out_spec] * 6,
        out_shape=[out_shape] * 6,
        scratch_shapes=[pltpu.VMEM(big, jnp.float32)] * 6
        + [pltpu.VMEM((B * _NUM_CLASSES, _LANES), jnp.float32)],
    )(X1, Y1, X2, Y2, SC, cf)
    res = outs[5]

    out6 = jnp.transpose(res[:, 0:6, 0:_MAX_DET], (0, 2, 1))
    valid_det = res[:, 6, 0].astype(jnp.int32)
    return out6, valid_det


@jax.jit
def kernel(images, predictions):
    B = predictions.shape[0]

    def _flat(a, pad_value):
        a = jnp.pad(a, ((0, 0), (0, _NPAD - _N)), constant_values=pad_value)
        return a.reshape(B * _NPAD)

    x1f = _flat(predictions[..., 0], 0.0)
    y1f = _flat(predictions[..., 1], 0.0)
    x2f = _flat(predictions[..., 2], 0.0)
    y2f = _flat(predictions[..., 3], 0.0)
    clsf = _flat(predictions[..., 4].astype(jnp.int32), _NUM_CLASSES)
    scf = _flat(predictions[..., 5], 0.0)

    ox1, oy1, ox2, oy2, osc, ocnt = _sc_compact(clsf, x1f, y1f, x2f, y2f, scf)

    shp = (B, _NUM_CLASSES * _SLOT_ROWS, _LANES)
    X1 = ox1.reshape(shp)
    Y1 = oy1.reshape(shp)
    X2 = ox2.reshape(shp)
    Y2 = oy2.reshape(shp)
    SCp = osc.reshape(shp)
    cnts = ocnt.reshape(B, _NUM_CLASSES, 16)[:, :, 0].astype(jnp.float32)
    cf = jnp.zeros((B, _LANES), jnp.float32).at[:, :_NUM_CLASSES].set(cnts)
    cf = cf.reshape(B, 1, _LANES)

    return _nms_from_compact(X1, Y1, X2, Y2, SCp, cf)


# speedup vs baseline: 1.6035x; 1.0006x over previous
import jax
import jax.numpy as jnp
from jax.experimental import pallas as pl
from jax.experimental.pallas import tpu as pltpu
from jax.experimental.pallas import tpu_sc as plsc

_NUM_CLASSES = 8
_CONF = 0.05
_IOU_T = 0.5
_MAX_DET = 100
_MAX_DET_PER_CLASS = 100

_N = 20000
_NPAD = 20480
_LANES = 128

_SLOT_ROWS = 168
_SLOT = _SLOT_ROWS * _LANES
_FAST_ROWS = 24
_CHUNK = 1280
_NCHUNKS = _NPAD // _CHUNK


def _compact_kernel(cls_hbm, x1_hbm, y1_hbm, x2_hbm, y2_hbm, sc_hbm,
                    ox1, oy1, ox2, oy2, osc, ocnt,
                    cls_v, px1, py1, px2, py2, psc,
                    cls_w, qx1, qy1, qx2, qy2, qsc,
                    st_x1, st_y1, st_x2, st_y2, st_sc, cnt_v,
                    sem_a, sem_b):
    b = jax.lax.axis_index("c")
    k = jax.lax.axis_index("s")

    @pl.when(k < _NUM_CLASSES)
    def _():
        myc = k

        def zbody(j, carry):
            st_sc[pl.ds(j * 16, 16)] = jnp.zeros((16,), jnp.float32)
            return carry

        jax.lax.fori_loop(0, (_SLOT + 16) // 16, zbody, jnp.int32(0))

        ones16 = jnp.full((16,), 1, jnp.int32)
        zeros16 = jnp.full((16,), 0, jnp.int32)
        lane16 = jnp.arange(16, dtype=jnp.int32)
        trash16 = lane16 + jnp.full((16,), _SLOT, jnp.int32)
        mycv = jnp.full((16,), myc, jnp.int32)
        sixteen16 = jnp.full((16,), 16, jnp.int32)

        bufs_a = (cls_v, px1, py1, px2, py2, psc)
        bufs_b = (cls_w, qx1, qy1, qx2, qy2, qsc)
        srcs = (cls_hbm, x1_hbm, y1_hbm, x2_hbm, y2_hbm, sc_hbm)

        def copies(t, bufs, sem):
            off = pl.multiple_of(b * _NPAD + t * _CHUNK, 8)
            return [pltpu.make_async_copy(s.at[pl.ds(off, _CHUNK)], d, sem)
                    for s, d in zip(srcs, bufs)]

        def issue(t, bufs, sem):
            for c in copies(t, bufs, sem):
                c.start()

        def wait(t, bufs, sem):
            for c in copies(t, bufs, sem):
                c.wait()

        def process(bufs, curv):
            cv, bx1, by1, bx2, by2, bsc = bufs

            def vec4(j, cur):
                for u in range(4):
                    i = j * 4 + u
                    v = cv[pl.ds(i * 16, 16)]
                    mask = v == mycv
                    pop = plsc.all_reduce_population_count(mask)
                    keys = jnp.where(mask, lane16, lane16 + sixteen16)
                    _, sval = plsc.sort_key_val(keys, lane16)
                    srcdst = jnp.where(lane16 < pop, cur + lane16, trash16)
                    plsc.store_scatter(cnt_v, [sval], srcdst)
                    dst = cnt_v[...]
                    plsc.store_scatter(st_x1, [dst], bx1[pl.ds(i * 16, 16)])
                    plsc.store_scatter(st_y1, [dst], by1[pl.ds(i * 16, 16)])
                    plsc.store_scatter(st_x2, [dst], bx2[pl.ds(i * 16, 16)])
                    plsc.store_scatter(st_y2, [dst], by2[pl.ds(i * 16, 16)])
                    plsc.store_scatter(st_sc, [dst], bsc[pl.ds(i * 16, 16)])
                    cur = cur + pop
                return cur

            return jax.lax.fori_loop(0, _CHUNK // 64, vec4, curv)

        issue(0, bufs_a, sem_a)

        def outer(t, curv):
            t2 = t * 2
            issue(t2 + 1, bufs_b, sem_b)
            wait(t2, bufs_a, sem_a)
            curv = process(bufs_a, curv)

            @pl.when(t < _NCHUNKS // 2 - 1)
            def _():
                issue(t2 + 2, bufs_a, sem_a)

            wait(t2 + 1, bufs_b, sem_b)
            curv = process(bufs_b, curv)
            return curv

        n_v = jax.lax.fori_loop(0, _NCHUNKS // 2, outer,
                                jnp.full((16,), 0, jnp.int32))
        npad_v = ((n_v + jnp.full((16,), 7, jnp.int32))
                  & jnp.full((16,), -8, jnp.int32))

        slot = pl.multiple_of((b * _NUM_CLASSES + myc) * _SLOT, 8)
        pltpu.sync_copy(st_x1.at[pl.ds(0, _SLOT)], ox1.at[pl.ds(slot, _SLOT)])
        pltpu.sync_copy(st_y1.at[pl.ds(0, _SLOT)], oy1.at[pl.ds(slot, _SLOT)])
        pltpu.sync_copy(st_x2.at[pl.ds(0, _SLOT)], ox2.at[pl.ds(slot, _SLOT)])
        pltpu.sync_copy(st_y2.at[pl.ds(0, _SLOT)], oy2.at[pl.ds(slot, _SLOT)])
        pltpu.sync_copy(st_sc.at[pl.ds(0, _SLOT)], osc.at[pl.ds(slot, _SLOT)])

        iota16 = jnp.arange(16, dtype=jnp.int32)
        cnt_v[...] = jnp.where(iota16 == 0, npad_v,
                               jnp.full((16,), 0, jnp.int32))
        coff = pl.multiple_of((b * _NUM_CLASSES + myc) * 16, 8)
        pltpu.sync_copy(cnt_v, ocnt.at[pl.ds(coff, 16)])


def _sc_compact(cls_f, x1f, y1f, x2f, y2f, scf):
    pay = jax.ShapeDtypeStruct((2 * _NUM_CLASSES * _SLOT,), jnp.float32)
    kfn = pl.kernel(
        _compact_kernel,
        mesh=plsc.VectorSubcoreMesh(core_axis_name="c", subcore_axis_name="s"),
        compiler_params=pltpu.CompilerParams(needs_layout_passes=False),
        out_type=[pay] * 5
        + [jax.ShapeDtypeStruct((2 * _NUM_CLASSES * 16,), jnp.int32)],
        scratch_types=([pltpu.VMEM((_CHUNK,), jnp.int32)]
                       + [pltpu.VMEM((_CHUNK,), jnp.float32)] * 5) * 2
        + [pltpu.VMEM((_SLOT + 16,), jnp.float32)] * 5
        + [pltpu.VMEM((16,), jnp.int32)]
        + [pltpu.SemaphoreType.DMA] * 2,
    )
    return kfn(cls_f, x1f, y1f, x2f, y2f, scf)


def _nms_kernel(x1, y1, x2, y2, sc, cf,
                sel_s, sel_y1, sel_x1, sel_y2, sel_x2, res,
                ny1, nx1, ny2, nx2, a2s, ss, mscr):
    NBC = 2 * _NUM_CLASSES
    lane = jax.lax.broadcasted_iota(jnp.int32, (1, _LANES), 1)
    tot = []
    for bc in range(NBC):
        b, c = divmod(bc, _NUM_CLASSES)
        tot.append(jnp.sum(jnp.where(lane == c, cf[b], 0.0))
                   .astype(jnp.int32))
    maxt = tot[0]
    for bc in range(1, NBC):
        maxt = jnp.maximum(maxt, tot[bc])

    neg = jnp.full((_NUM_CLASSES, _LANES), -jnp.inf, jnp.float32)
    zeros = jnp.zeros((_NUM_CLASSES, _LANES), jnp.float32)
    for b in range(2):
        sel_s[b] = neg
        sel_y1[b] = zeros
        sel_x1[b] = zeros
        sel_y2[b] = zeros
        sel_x2[b] = zeros

    def run(R):
        flat = (jax.lax.broadcasted_iota(jnp.int32, (R, _LANES), 0) * _LANES
                + jax.lax.broadcasted_iota(jnp.int32, (R, _LANES), 1))
        for bc in range(NBC):
            b, c = divmod(bc, _NUM_CLASSES)
            ib = c * _SLOT_ROWS
            base = bc * _SLOT_ROWS
            ny1v = y1[b, pl.ds(ib, R), :] / 512.0
            nx1v = x1[b, pl.ds(ib, R), :] / 512.0
            ny2v = y2[b, pl.ds(ib, R), :] / 512.0
            nx2v = x2[b, pl.ds(ib, R), :] / 512.0
            ny1[pl.ds(base, R), :] = ny1v
            nx1[pl.ds(base, R), :] = nx1v
            ny2[pl.ds(base, R), :] = ny2v
            nx2[pl.ds(base, R), :] = nx2v
            a2s[pl.ds(base, R), :] = (ny2v - ny1v) * (nx2v - nx1v)
            scv = sc[b, pl.ds(ib, R), :]
            ss[pl.ds(base, R), :] = jnp.where(
                (flat < tot[bc]) & (scv > _CONF), scv, -jnp.inf)

        m_init = tuple(jnp.max(ss[pl.ds(bc * _SLOT_ROWS, R), :])
                       for bc in range(NBC))

        def cond(carry):
            step = carry[0]
            ms = carry[1:]
            any_m = ms[0]
            for bc in range(1, NBC):
                any_m = jnp.maximum(any_m, ms[bc])
            return (step < _MAX_DET_PER_CLASS) & (any_m > _CONF)

        def body(carry):
            step = carry[0]
            ms = carry[1:]
            act = [ms[bc] > _CONF for bc in range(NBC)]
            sv = [ss[pl.ds(bc * _SLOT_ROWS, R), :] for bc in range(NBC)]
            idx = [None] * NBC
            for bc in range(NBC):
                eq = sv[bc] == ms[bc]
                idx[bc] = jnp.min(jnp.where(eq, flat, jnp.int32(2 ** 30)))
            row, colmask = [None] * NBC, [None] * NBC
            for bc in range(NBC):
                i = jnp.where(act[bc], idx[bc], 0)
                row[bc] = bc * _SLOT_ROWS + i // _LANES
                colmask[bc] = lane == (i % _LANES)
            by1 = [None] * NBC
            bx1 = [None] * NBC
            by2 = [None] * NBC
            bx2 = [None] * NBC
            for bc in range(NBC):
                by1[bc] = jnp.sum(jnp.where(colmask[bc], ny1[pl.ds(row[bc], 1), :], 0.0))
                bx1[bc] = jnp.sum(jnp.where(colmask[bc], nx1[pl.ds(row[bc], 1), :], 0.0))
                by2[bc] = jnp.sum(jnp.where(colmask[bc], ny2[pl.ds(row[bc], 1), :], 0.0))
                bx2[bc] = jnp.sum(jnp.where(colmask[bc], nx2[pl.ds(row[bc], 1), :], 0.0))
            for bc in range(NBC):
                b, c = divmod(bc, _NUM_CLASSES)
                lm = (lane == step) & act[bc]
                sel_s[b, pl.ds(c, 1), :] = jnp.where(lm, ms[bc], sel_s[b, pl.ds(c, 1), :])
                sel_y1[b, pl.ds(c, 1), :] = jnp.where(lm, by1[bc], sel_y1[b, pl.ds(c, 1), :])
                sel_x1[b, pl.ds(c, 1), :] = jnp.where(lm, bx1[bc], sel_x1[b, pl.ds(c, 1), :])
                sel_y2[b, pl.ds(c, 1), :] = jnp.where(lm, by2[bc], sel_y2[b, pl.ds(c, 1), :])
                sel_x2[b, pl.ds(c, 1), :] = jnp.where(lm, bx2[bc], sel_x2[b, pl.ds(c, 1), :])
            new_ms = []
            for bc in range(NBC):
                base = bc * _SLOT_ROWS
                yy1 = jnp.maximum(by1[bc], ny1[pl.ds(base, R), :])
                xx1 = jnp.maximum(bx1[bc], nx1[pl.ds(base, R), :])
                yy2 = jnp.minimum(by2[bc], ny2[pl.ds(base, R), :])
                xx2 = jnp.minimum(bx2[bc], nx2[pl.ds(base, R), :])
                inter = jnp.maximum(yy2 - yy1, 0.0) * jnp.maximum(xx2 - xx1, 0.0)
                a1 = (by2[bc] - by1[bc]) * (bx2[bc] - bx1[bc])
                iou = inter / (a1 + a2s[pl.ds(base, R), :] - inter + 1e-8)
                snew = jnp.where(iou > _IOU_T, -jnp.inf, sv[bc])
                ss[pl.ds(base, R), :] = snew
                new_ms.append(jnp.max(snew))
            return (step + 1,) + tuple(new_ms)

        jax.lax.while_loop(cond, body, (jnp.int32(0),) + m_init)

    @pl.when(maxt <= _FAST_ROWS * _LANES)
    def _():
        run(_FAST_ROWS)

    @pl.when(maxt > _FAST_ROWS * _LANES)
    def _():
        run(_SLOT_ROWS)

    crow = jax.lax.broadcasted_iota(jnp.int32, (_NUM_CLASSES, _LANES), 0)
    lane8 = jax.lax.broadcasted_iota(jnp.int32, (_NUM_CLASSES, _LANES), 1)
    validlane = lane8 < _MAX_DET_PER_CLASS
    flat8 = jnp.where(validlane, crow * _MAX_DET_PER_CLASS + lane8,
                      jnp.int32(2 ** 30))

    for b in range(2):
        mscr[pl.ds(b * _NUM_CLASSES, _NUM_CLASSES), :] = jnp.where(
            validlane, sel_s[b], -jnp.inf)
        res[b] = jnp.zeros((_NUM_CLASSES, _LANES), jnp.float32)

    mm = tuple(jnp.max(mscr[pl.ds(b * _NUM_CLASSES, _NUM_CLASSES), :])
               for b in range(2))

    def mcond(carry):
        step = carry[0]
        return (step < _MAX_DET) & (jnp.maximum(carry[1], carry[2]) > _CONF)

    def mbody(carry):
        step, ma0, ma1, cn0, cn1 = carry
        ms2 = (ma0, ma1)
        cns = [cn0, cn1]
        new_ms = []
        for b in range(2):
            m = ms2[b]
            act = m > _CONF
            sv = mscr[pl.ds(b * _NUM_CLASSES, _NUM_CLASSES), :]
            eq = sv == m
            fidx = jnp.min(jnp.where(eq, flat8, jnp.int32(2 ** 30)))
            fidx = jnp.where(act, fidx, 0)
            c = fidx // _MAX_DET_PER_CLASS
            j = fidx % _MAX_DET_PER_CLASS
            mask = (crow == c) & (lane8 == j)
            by1 = jnp.sum(jnp.where(mask, sel_y1[b], 0.0))
            bx1 = jnp.sum(jnp.where(mask, sel_x1[b], 0.0))
            by2 = jnp.sum(jnp.where(mask, sel_y2[b], 0.0))
            bx2 = jnp.sum(jnp.where(mask, sel_x2[b], 0.0))

            lm = (lane == step) & act
            res[b, pl.ds(0, 1), :] = jnp.where(lm, bx1 * 512.0, res[b, pl.ds(0, 1), :])
            res[b, pl.ds(1, 1), :] = jnp.where(lm, by1 * 512.0, res[b, pl.ds(1, 1), :])
            res[b, pl.ds(2, 1), :] = jnp.where(lm, bx2 * 512.0, res[b, pl.ds(2, 1), :])
            res[b, pl.ds(3, 1), :] = jnp.where(lm, by2 * 512.0, res[b, pl.ds(3, 1), :])
            res[b, pl.ds(4, 1), :] = jnp.where(lm, c.astype(jnp.float32), res[b, pl.ds(4, 1), :])
            res[b, pl.ds(5, 1), :] = jnp.where(lm, m, res[b, pl.ds(5, 1), :])

            snew = jnp.where(mask & act, -jnp.inf, sv)
            mscr[pl.ds(b * _NUM_CLASSES, _NUM_CLASSES), :] = snew
            new_ms.append(jnp.max(snew))
            cns[b] = cns[b] + jnp.where(act, jnp.int32(1), jnp.int32(0))
        return (step + 1, new_ms[0], new_ms[1], cns[0], cns[1])

    fin = jax.lax.while_loop(
        mcond, mbody, (jnp.int32(0), mm[0], mm[1], jnp.int32(0), jnp.int32(0)))
    for b in range(2):
        res[b, pl.ds(6, 1), :] = jnp.where(lane == 0,
                                           fin[3 + b].astype(jnp.float32),
                                           res[b, pl.ds(6, 1), :])


def _nms_from_compact(X1, Y1, X2, Y2, SC, cf):
    B = X1.shape[0]
    pay_spec = pl.BlockSpec((B, _NUM_CLASSES * _SLOT_ROWS, _LANES),
                            lambda i: (0, 0, 0))
    cf_spec = pl.BlockSpec((B, 1, _LANES), lambda i: (0, 0, 0))
    out_spec = pl.BlockSpec((B, _NUM_CLASSES, _LANES), lambda i: (0, 0, 0))
    out_shape = jax.ShapeDtypeStruct((B, _NUM_CLASSES, _LANES), jnp.float32)
    big = (B * _NUM_CLASSES * _SLOT_ROWS, _LANES)

    outs = pl.pallas_call(
        _nms_kernel,
        grid=(1,),
        in_specs=[pay_spec] * 5 + [cf_spec],
        out_specs=[out_spec] * 6,
        out_shape=[out_shape] * 6,
        scratch_shapes=[pltpu.VMEM(big, jnp.float32)] * 6
        + [pltpu.VMEM((B * _NUM_CLASSES, _LANES), jnp.float32)],
    )(X1, Y1, X2, Y2, SC, cf)
    res = outs[5]

    out6 = jnp.transpose(res[:, 0:6, 0:_MAX_DET], (0, 2, 1))
    valid_det = res[:, 6, 0].astype(jnp.int32)
    return out6, valid_det


@jax.jit
def kernel(images, predictions):
    B = predictions.shape[0]

    def _flat(a, pad_value):
        a = jnp.pad(a, ((0, 0), (0, _NPAD - _N)), constant_values=pad_value)
        return a.reshape(B * _NPAD)

    x1f = _flat(predictions[..., 0], 0.0)
    y1f = _flat(predictions[..., 1], 0.0)
    x2f = _flat(predictions[..., 2], 0.0)
    y2f = _flat(predictions[..., 3], 0.0)
    clsf = _flat(predictions[..., 4].astype(jnp.int32), _NUM_CLASSES)
    scf = _flat(predictions[..., 5], 0.0)

    ox1, oy1, ox2, oy2, osc, ocnt = _sc_compact(clsf, x1f, y1f, x2f, y2f, scf)

    shp = (B, _NUM_CLASSES * _SLOT_ROWS, _LANES)
    X1 = ox1.reshape(shp)
    Y1 = oy1.reshape(shp)
    X2 = ox2.reshape(shp)
    Y2 = oy2.reshape(shp)
    SCp = osc.reshape(shp)
    cnts = ocnt.reshape(B, _NUM_CLASSES, 16)[:, :, 0].astype(jnp.float32)
    cf = jnp.zeros((B, _LANES), jnp.float32).at[:, :_NUM_CLASSES].set(cnts)
    cf = cf.reshape(B, 1, _LANES)

    return _nms_from_compact(X1, Y1, X2, Y2, SCp, cf)
